# Initial kernel scaffold; baseline (speedup 1.0000x reference)
import jax
import jax.numpy as jnp
from jax.experimental import pallas as pl


def kernel(V, E, theta, params, senders, receivers, real_node_indices):
    raise NotImplementedError("write your pallas kernel here")


# speedup vs baseline: 2.5708x; 2.5708x over previous
import functools

import jax
import jax.numpy as jnp
from jax import lax
from jax.experimental import pallas as pl
from jax.experimental.pallas import tpu as pltpu
from jax.experimental.pallas import tpu_sc as plsc

N_NODES = 10000
N_EDGES = 160000
LAT = 128
CH = 128
N_CHUNKS = N_EDGES // CH
NC, NS = 2, 16
NW = NC * NS
BLK_N = 1000
BLK_E = 1000
HALF = LAT // 2


def _celu(x):
    return jnp.where(x > 0, x, jnp.exp(jnp.minimum(x, 0.0)) - 1.0)


def _ln(x, g, beta):
    mu = jnp.mean(x, axis=-1, keepdims=True)
    d = x - mu
    var = jnp.mean(d * d, axis=-1, keepdims=True)
    return d * lax.rsqrt(var + 1e-6) * g + beta


def _mlp3(x, W0, b0, W1, b1, W2, b2, g, beta):
    h = _celu(jnp.dot(x, W0, preferred_element_type=jnp.float32) + b0)
    h = _celu(jnp.dot(h, W1, preferred_element_type=jnp.float32) + b1)
    h = jnp.dot(h, W2, preferred_element_type=jnp.float32) + b2
    return _ln(h, g, beta)


def _enc_nodes_body(v, W0, b0, W1, b1, W2, b2, g, beta, wr, ws, vl_o, pr_o, ps_o):
    vl = _mlp3(v[...], W0[...], b0[...], W1[...], b1[...], W2[...], b2[...],
               g[...], beta[...])
    vl_o[...] = vl
    pr_o[...] = jnp.dot(vl, wr[...], preferred_element_type=jnp.float32)
    ps_o[...] = jnp.dot(vl, ws[...], preferred_element_type=jnp.float32)


def _enc_edges_body(e, W0, b0, W1, b1, W2, b2, g, beta, el_o):
    el_o[...] = _mlp3(e[...], W0[...], b0[...], W1[...], b1[...], W2[...],
                      b2[...], g[...], beta[...])


def _edge_tail_body(el, g1, g2, W0a, b0, W1, b1, W2, b2, g, beta, m_o, eln_o):
    x = el[...]
    h = _celu(jnp.dot(x, W0a[...], preferred_element_type=jnp.float32)
              + g1[...] + g2[...] + b0[...])
    h = _celu(jnp.dot(h, W1[...], preferred_element_type=jnp.float32) + b1[...])
    h = jnp.dot(h, W2[...], preferred_element_type=jnp.float32) + b2[...]
    m = _ln(h, g[...], beta[...])
    m_o[...] = m
    eln_o[...] = x + m


def _node_tail_body(vl, a, b, W0a, W0b, b0, W1, b1, W2, b2, g, beta, wr, ws,
                    vln_o, pr_o, ps_o):
    x = vl[...]
    s = a[...] - b[...]
    h = _celu(jnp.dot(x, W0a[...], preferred_element_type=jnp.float32)
              + jnp.dot(s, W0b[...], preferred_element_type=jnp.float32)
              + b0[...])
    h = _celu(jnp.dot(h, W1[...], preferred_element_type=jnp.float32) + b1[...])
    h = jnp.dot(h, W2[...], preferred_element_type=jnp.float32) + b2[...]
    vln = x + _ln(h, g[...], beta[...])
    vln_o[...] = vln
    if pr_o is not None:
        pr_o[...] = jnp.dot(vln, wr[...], preferred_element_type=jnp.float32)
        ps_o[...] = jnp.dot(vln, ws[...], preferred_element_type=jnp.float32)


def _node_tail_last_body(vl, a, b, W0a, W0b, b0, W1, b1, W2, b2, g, beta, vln_o):
    _node_tail_body(vl, a, b, W0a, W0b, b0, W1, b1, W2, b2, g, beta, None, None,
                    vln_o, None, None)


def _theta_body(t, W0, b0, W1, b1, W2, b2, g, beta, dW0t, db0, out):
    h = _celu(jnp.dot(t[...], W0[...], preferred_element_type=jnp.float32) + b0[...])
    h = _celu(jnp.dot(h, W1[...], preferred_element_type=jnp.float32) + b1[...])
    h = jnp.dot(h, W2[...], preferred_element_type=jnp.float32) + b2[...]
    zt = _ln(h, g[...], beta[...])
    rows = []
    for d in range(3):
        wd = dW0t[d * LAT:(d + 1) * LAT, :]
        rows.append(jnp.dot(zt, wd, preferred_element_type=jnp.float32)
                    + db0[d:d + 1, :])
    out[...] = jnp.concatenate(rows, axis=0)


def _final_body(vl, inc, mask, g_f, beta_f, dW0z, dbe, dW1, db1, dW2, db2, out):
    m = mask[...]
    z = jnp.concatenate([vl[...] * m, inc[...] * m], axis=1)
    zl = _ln(z, g_f[...], beta_f[...])
    cols = []
    for d in range(3):
        h = _celu(jnp.dot(zl, dW0z[d * 2 * LAT:(d + 1) * 2 * LAT, :],
                          preferred_element_type=jnp.float32) + dbe[d:d + 1, :])
        h = _celu(jnp.dot(h, dW1[d * LAT:(d + 1) * LAT, :],
                          preferred_element_type=jnp.float32) + db1[d:d + 1, :])
        cols.append(jnp.dot(h, dW2[:, d:d + 1],
                            preferred_element_type=jnp.float32))
    out[...] = jnp.concatenate(cols, axis=1) + db2[...]


def _full(shape):
    return pl.BlockSpec(shape, lambda i: (0,) * len(shape))


def _rows(blk, width):
    return pl.BlockSpec((blk, width), lambda i: (i, 0))


def _tc_enc_nodes(V, p, wr, ws):
    (W0, b0), (W1, b1), (W2, b2), (g, beta) = p
    n = N_NODES // BLK_N
    args = [V, W0, b0.reshape(1, -1), W1, b1.reshape(1, -1), W2,
            b2.reshape(1, -1), g.reshape(1, -1), beta.reshape(1, -1), wr, ws]
    specs = [_rows(BLK_N, LAT)] + [_full(a.shape) for a in args[1:]]
    return pl.pallas_call(
        _enc_nodes_body,
        grid=(n,),
        in_specs=specs,
        out_specs=[_rows(BLK_N, LAT)] * 3,
        out_shape=[jax.ShapeDtypeStruct((N_NODES, LAT), jnp.float32)] * 3,
    )(*args)


def _tc_enc_edges(E, p):
    (W0, b0), (W1, b1), (W2, b2), (g, beta) = p
    n = N_EDGES // BLK_E
    args = [E, W0, b0.reshape(1, -1), W1, b1.reshape(1, -1), W2,
            b2.reshape(1, -1), g.reshape(1, -1), beta.reshape(1, -1)]
    specs = [_rows(BLK_E, E.shape[1])] + [_full(a.shape) for a in args[1:]]
    return pl.pallas_call(
        _enc_edges_body,
        grid=(n,),
        in_specs=specs,
        out_specs=_rows(BLK_E, LAT),
        out_shape=jax.ShapeDtypeStruct((N_EDGES, LAT), jnp.float32),
    )(*args)


def _tc_edge_tail(El, G1, G2, p):
    (W0, b0), (W1, b1), (W2, b2), (g, beta) = p
    W0a = W0[:LAT, :]
    n = N_EDGES // BLK_E
    args = [El, G1, G2, W0a, b0.reshape(1, -1), W1, b1.reshape(1, -1), W2,
            b2.reshape(1, -1), g.reshape(1, -1), beta.reshape(1, -1)]
    specs = [_rows(BLK_E, LAT)] * 3 + [_full(a.shape) for a in args[3:]]
    return pl.pallas_call(
        _edge_tail_body,
        grid=(n,),
        in_specs=specs,
        out_specs=[_rows(BLK_E, LAT)] * 2,
        out_shape=[jax.ShapeDtypeStruct((N_EDGES, LAT), jnp.float32)] * 2,
    )(*args)


def _tc_node_tail(Vl, A, B, p, wr=None, ws=None):
    (W0, b0), (W1, b1), (W2, b2), (g, beta) = p
    W0a, W0b = W0[:LAT, :], W0[LAT:, :]
    n = N_NODES // BLK_N
    args = [Vl, A, B, W0a, W0b, b0.reshape(1, -1), W1, b1.reshape(1, -1), W2,
            b2.reshape(1, -1), g.reshape(1, -1), beta.reshape(1, -1)]
    specs = [_rows(BLK_N, LAT)] * 3 + [_full(a.shape) for a in args[3:]]
    if wr is not None:
        args += [wr, ws]
        specs += [_full(wr.shape), _full(ws.shape)]
        return pl.pallas_call(
            _node_tail_body,
            grid=(n,),
            in_specs=specs,
            out_specs=[_rows(BLK_N, LAT)] * 3,
            out_shape=[jax.ShapeDtypeStruct((N_NODES, LAT), jnp.float32)] * 3,
        )(*args)
    return pl.pallas_call(
        _node_tail_last_body,
        grid=(n,),
        in_specs=specs,
        out_specs=_rows(BLK_N, LAT),
        out_shape=jax.ShapeDtypeStruct((N_NODES, LAT), jnp.float32),
    )(*args)


def _tc_theta(theta2d, p, dW0t, db0):
    (W0, b0), (W1, b1), (W2, b2), (g, beta) = p
    args = [theta2d, W0, b0.reshape(1, -1), W1, b1.reshape(1, -1), W2,
            b2.reshape(1, -1), g.reshape(1, -1), beta.reshape(1, -1), dW0t, db0]
    return pl.pallas_call(
        _theta_body,
        grid=(1,),
        in_specs=[_full(a.shape) for a in args],
        out_specs=_full((3, LAT)),
        out_shape=jax.ShapeDtypeStruct((3, LAT), jnp.float32),
    )(*args)


def _tc_final(Vl, Inc, mask, g_f, beta_f, dW0z, dbe, dW1, db1, dW2, db2):
    n = N_NODES // BLK_N
    args = [Vl, Inc, mask, g_f.reshape(1, -1), beta_f.reshape(1, -1),
            dW0z, dbe, dW1, db1, dW2, db2]
    specs = ([_rows(BLK_N, LAT)] * 2 + [_rows(BLK_N, 1)]
             + [_full(a.shape) for a in args[3:]])
    return pl.pallas_call(
        _final_body,
        grid=(n,),
        in_specs=specs,
        out_specs=_rows(BLK_N, 3),
        out_shape=jax.ShapeDtypeStruct((N_NODES, 3), jnp.float32),
    )(*args)


_MESH = plsc.VectorSubcoreMesh(core_axis_name="c", subcore_axis_name="s")

_G_BASE = N_CHUNKS // NW
_G_EXTRA = N_CHUNKS - _G_BASE * NW
_S_BASE = N_CHUNKS // NS
_S_EXTRA = N_CHUNKS - _S_BASE * NS
_HALF_N = N_NODES // NC
_TAB_ROWS = _HALF_N + 8
_INIT_R = 312
_OUT_R = 312


def _gather_body(pr_hbm, ps_hbm, recv_hbm, send_hbm, g1_hbm, g2_hbm,
                 idxr, idxs, bufr, bufs, sem1, sem2):
    c = lax.axis_index("c")
    s = lax.axis_index("s")
    w = s * NC + c
    nw = jnp.where(w < _G_EXTRA, _G_BASE + 1, _G_BASE)
    start = _G_BASE * w + jnp.minimum(w, _G_EXTRA)

    def body(i, carry):
        off = (start + i) * CH
        pltpu.sync_copy(recv_hbm.at[pl.ds(off, CH)], idxr)
        pltpu.sync_copy(send_hbm.at[pl.ds(off, CH)], idxs)
        d1 = pltpu.async_copy(pr_hbm.at[idxr], bufr, sem1)
        d2 = pltpu.async_copy(ps_hbm.at[idxs], bufs, sem2)
        d1.wait()
        d2.wait()
        pltpu.sync_copy(bufr, g1_hbm.at[pl.ds(off, CH)])
        pltpu.sync_copy(bufs, g2_hbm.at[pl.ds(off, CH)])
        return carry

    lax.fori_loop(0, nw, body, 0)


@functools.partial(
    pl.kernel,
    out_type=[jax.ShapeDtypeStruct((N_EDGES, LAT), jnp.float32)] * 2,
    mesh=_MESH,
    scratch_types=[
        pltpu.VMEM((CH,), jnp.int32),
        pltpu.VMEM((CH,), jnp.int32),
        pltpu.VMEM((CH, LAT), jnp.float32),
        pltpu.VMEM((CH, LAT), jnp.float32),
        pltpu.SemaphoreType.DMA,
        pltpu.SemaphoreType.DMA,
    ],
)
def _sc_gather(pr_hbm, ps_hbm, recv_hbm, send_hbm, g1_hbm, g2_hbm,
               idxr, idxs, bufr, bufs, sem1, sem2):
    _gather_body(pr_hbm, ps_hbm, recv_hbm, send_hbm, g1_hbm, g2_hbm,
                 idxr, idxs, bufr, bufs, sem1, sem2)


def _tab_init_all(zeros_hbm, table, s):
    for t in range(NS):
        @pl.when(s == t)
        def _():
            r0 = t * _INIT_R
            nr = _TAB_ROWS - 15 * _INIT_R if t == 15 else _INIT_R
            pltpu.sync_copy(zeros_hbm.at[pl.ds(r0, nr)], table.at[pl.ds(r0, nr)])


def _clamp_local(idx_ref, c):
    base = c * _HALF_N
    for j in range(CH // 16):
        v = idx_ref[pl.ds(j * 16, 16)]
        local = v - base
        ok = (local >= 0) & (local < _HALF_N)
        idx_ref[pl.ds(j * 16, 16)] = jnp.where(ok, local, _HALF_N)


def _tab_writeout(table, out_hbm, c, s):
    for t in range(NS):
        @pl.when(s == t)
        def _():
            r0 = t * _OUT_R
            nr = _HALF_N - 15 * _OUT_R if t == 15 else _OUT_R
            pltpu.sync_copy(table.at[pl.ds(r0, nr)],
                            out_hbm.at[pl.ds(c * _HALF_N + r0, nr)])


def _scatter2_body(m_hbm, recv_hbm, send_hbm, zeros_hbm, a_hbm, b_hbm,
                   idxr, idxs, buf, tabA, tabB):
    c = lax.axis_index("c")
    s = lax.axis_index("s")
    _tab_init_all(zeros_hbm, tabA, s)
    _tab_init_all(zeros_hbm, tabB, s)
    ns = jnp.where(s < _S_EXTRA, _S_BASE + 1, _S_BASE)
    start = _S_BASE * s + jnp.minimum(s, _S_EXTRA)
    plsc.subcore_barrier()

    def body(i, carry):
        off = (start + i) * CH
        pltpu.sync_copy(recv_hbm.at[pl.ds(off, CH)], idxr)
        pltpu.sync_copy(send_hbm.at[pl.ds(off, CH)], idxs)
        pltpu.sync_copy(m_hbm.at[pl.ds(off, CH)], buf)
        _clamp_local(idxr, c)
        _clamp_local(idxs, c)
        pltpu.sync_copy(buf, tabA.at[idxr], add=True)
        pltpu.sync_copy(buf, tabB.at[idxs], add=True)
        return carry

    lax.fori_loop(0, ns, body, 0)
    plsc.subcore_barrier()
    _tab_writeout(tabA, a_hbm, c, s)
    _tab_writeout(tabB, b_hbm, c, s)


@functools.partial(
    pl.kernel,
    out_type=[jax.ShapeDtypeStruct((N_NODES, LAT), jnp.float32)] * 2,
    mesh=_MESH,
    scratch_types=[
        pltpu.VMEM((CH,), jnp.int32),
        pltpu.VMEM((CH,), jnp.int32),
        pltpu.VMEM((CH, LAT), jnp.float32),
        pltpu.VMEM_SHARED((_TAB_ROWS, LAT), jnp.float32),
        pltpu.VMEM_SHARED((_TAB_ROWS, LAT), jnp.float32),
    ],
)
def _sc_scatter2(m_hbm, recv_hbm, send_hbm, zeros_hbm, a_hbm, b_hbm,
                 idxr, idxs, buf, tabA, tabB):
    _scatter2_body(m_hbm, recv_hbm, send_hbm, zeros_hbm, a_hbm, b_hbm,
                   idxr, idxs, buf, tabA, tabB)


def _scatter1_body(m_hbm, recv_hbm, zeros_hbm, a_hbm, idxr, buf, tabA):
    c = lax.axis_index("c")
    s = lax.axis_index("s")
    _tab_init_all(zeros_hbm, tabA, s)
    ns = jnp.where(s < _S_EXTRA, _S_BASE + 1, _S_BASE)
    start = _S_BASE * s + jnp.minimum(s, _S_EXTRA)
    plsc.subcore_barrier()

    def body(i, carry):
        off = (start + i) * CH
        pltpu.sync_copy(recv_hbm.at[pl.ds(off, CH)], idxr)
        pltpu.sync_copy(m_hbm.at[pl.ds(off, CH)], buf)
        _clamp_local(idxr, c)
        pltpu.sync_copy(buf, tabA.at[idxr], add=True)
        return carry

    lax.fori_loop(0, ns, body, 0)
    plsc.subcore_barrier()
    _tab_writeout(tabA, a_hbm, c, s)


@functools.partial(
    pl.kernel,
    out_type=jax.ShapeDtypeStruct((N_NODES, LAT), jnp.float32),
    mesh=_MESH,
    scratch_types=[
        pltpu.VMEM((CH,), jnp.int32),
        pltpu.VMEM((CH, LAT), jnp.float32),
        pltpu.VMEM_SHARED((_TAB_ROWS, LAT), jnp.float32),
    ],
)
def _sc_scatter1(m_hbm, recv_hbm, zeros_hbm, a_hbm, idxr, buf, tabA):
    _scatter1_body(m_hbm, recv_hbm, zeros_hbm, a_hbm, idxr, buf, tabA)


def kernel(V, E, theta, params, senders, receivers, real_node_indices):
    zeros_tab = jnp.zeros((_TAB_ROWS, LAT), jnp.float32)
    mask = real_node_indices.astype(jnp.float32).reshape(N_NODES, 1)
    theta2d = theta.reshape(1, -1)

    mp = params['mp']
    wr = [blk['edge'][0][0][LAT:2 * LAT, :] for blk in mp]
    ws = [blk['edge'][0][0][2 * LAT:, :] for blk in mp]

    Vl, Pr, Ps = _tc_enc_nodes(V, params['node_enc'], wr[0], ws[0])
    El = _tc_enc_edges(E, params['edge_enc'])

    K = len(mp)
    for k in range(K):
        G1, G2 = _sc_gather(Pr, Ps, receivers, senders)
        M, El = _tc_edge_tail(El, G1, G2, mp[k]['edge'])
        A, B = _sc_scatter2(M, receivers, senders, zeros_tab)
        if k + 1 < K:
            Vl, Pr, Ps = _tc_node_tail(Vl, A, B, mp[k]['node'],
                                       wr[k + 1], ws[k + 1])
        else:
            Vl = _tc_node_tail(Vl, A, B, mp[k]['node'])

    Inc = _sc_scatter1(El, receivers, zeros_tab)

    dec = params['dec']
    dW0t = jnp.concatenate([dec[d][0][0][:LAT, :] for d in range(3)], axis=0)
    dW0z = jnp.concatenate([dec[d][0][0][LAT:, :] for d in range(3)], axis=0)
    db0 = jnp.stack([dec[d][0][1] for d in range(3)])
    dW1 = jnp.concatenate([dec[d][1][0] for d in range(3)], axis=0)
    db1 = jnp.stack([dec[d][1][1] for d in range(3)])
    dW2 = jnp.concatenate([dec[d][2][0] for d in range(3)], axis=1)
    db2 = jnp.stack([dec[d][2][1] for d in range(3)]).reshape(1, 3)

    dbe = _tc_theta(theta2d, params['theta_enc'], dW0t, db0)
    g_f, beta_f = params['final_ln']
    return _tc_final(Vl, Inc, mask, g_f, beta_f, dW0z, dbe, dW1, db1, dW2, db2)


# speedup vs baseline: 2.9651x; 1.1534x over previous
import functools

import jax
import jax.numpy as jnp
from jax import lax
from jax.experimental import pallas as pl
from jax.experimental.pallas import tpu as pltpu
from jax.experimental.pallas import tpu_sc as plsc

N_NODES = 10000
N_EDGES = 160000
LAT = 128
CH = 128
N_CHUNKS = N_EDGES // CH
NC, NS = 2, 16
NW = NC * NS
BLK_N = 1000
BLK_E = 1000
HALF = LAT // 2


def _celu(x):
    return jnp.where(x > 0, x, jnp.exp(jnp.minimum(x, 0.0)) - 1.0)


def _ln(x, g, beta):
    mu = jnp.mean(x, axis=-1, keepdims=True)
    d = x - mu
    var = jnp.mean(d * d, axis=-1, keepdims=True)
    return d * lax.rsqrt(var + 1e-6) * g + beta


def _mlp3(x, W0, b0, W1, b1, W2, b2, g, beta):
    h = _celu(jnp.dot(x, W0, preferred_element_type=jnp.float32) + b0)
    h = _celu(jnp.dot(h, W1, preferred_element_type=jnp.float32) + b1)
    h = jnp.dot(h, W2, preferred_element_type=jnp.float32) + b2
    return _ln(h, g, beta)


def _enc_nodes_body(v, W0, b0, W1, b1, W2, b2, g, beta, wr, ws, vl_o, pr_o, ps_o):
    vl = _mlp3(v[...], W0[...], b0[...], W1[...], b1[...], W2[...], b2[...],
               g[...], beta[...])
    vl_o[...] = vl
    pr_o[...] = jnp.dot(vl, wr[...], preferred_element_type=jnp.float32)
    ps_o[...] = jnp.dot(vl, ws[...], preferred_element_type=jnp.float32)


def _enc_edges_body(e, W0, b0, W1, b1, W2, b2, g, beta, el_o):
    el_o[...] = _mlp3(e[...], W0[...], b0[...], W1[...], b1[...], W2[...],
                      b2[...], g[...], beta[...])


def _edge_tail_body(el, g1, g2, W0a, b0, W1, b1, W2, b2, g, beta, m_o, eln_o):
    x = el[...]
    h = _celu(jnp.dot(x, W0a[...], preferred_element_type=jnp.float32)
              + g1[...] + g2[...] + b0[...])
    h = _celu(jnp.dot(h, W1[...], preferred_element_type=jnp.float32) + b1[...])
    h = jnp.dot(h, W2[...], preferred_element_type=jnp.float32) + b2[...]
    m = _ln(h, g[...], beta[...])
    m_o[...] = m
    eln_o[...] = x + m


def _node_tail_body(vl, a, b, W0a, W0b, b0, W1, b1, W2, b2, g, beta, wr, ws,
                    vln_o, pr_o, ps_o):
    x = vl[...]
    s = a[...] - b[...]
    h = _celu(jnp.dot(x, W0a[...], preferred_element_type=jnp.float32)
              + jnp.dot(s, W0b[...], preferred_element_type=jnp.float32)
              + b0[...])
    h = _celu(jnp.dot(h, W1[...], preferred_element_type=jnp.float32) + b1[...])
    h = jnp.dot(h, W2[...], preferred_element_type=jnp.float32) + b2[...]
    vln = x + _ln(h, g[...], beta[...])
    vln_o[...] = vln
    if pr_o is not None:
        pr_o[...] = jnp.dot(vln, wr[...], preferred_element_type=jnp.float32)
        ps_o[...] = jnp.dot(vln, ws[...], preferred_element_type=jnp.float32)


def _node_tail_last_body(vl, a, b, W0a, W0b, b0, W1, b1, W2, b2, g, beta, vln_o):
    _node_tail_body(vl, a, b, W0a, W0b, b0, W1, b1, W2, b2, g, beta, None, None,
                    vln_o, None, None)


def _theta_body(t, W0, b0, W1, b1, W2, b2, g, beta, dW0t, db0, out):
    h = _celu(jnp.dot(t[...], W0[...], preferred_element_type=jnp.float32) + b0[...])
    h = _celu(jnp.dot(h, W1[...], preferred_element_type=jnp.float32) + b1[...])
    h = jnp.dot(h, W2[...], preferred_element_type=jnp.float32) + b2[...]
    zt = _ln(h, g[...], beta[...])
    rows = []
    for d in range(3):
        wd = dW0t[d * LAT:(d + 1) * LAT, :]
        rows.append(jnp.dot(zt, wd, preferred_element_type=jnp.float32)
                    + db0[d:d + 1, :])
    out[...] = jnp.concatenate(rows, axis=0)


def _final_body(vl, inc0, a1, a2, mask, g_f, beta_f, dW0z, dbe, dW1, db1, dW2,
                db2, out):
    m = mask[...]
    inc = inc0[...] + a1[...] + a2[...]
    z = jnp.concatenate([vl[...] * m, inc * m], axis=1)
    zl = _ln(z, g_f[...], beta_f[...])
    cols = []
    for d in range(3):
        h = _celu(jnp.dot(zl, dW0z[d * 2 * LAT:(d + 1) * 2 * LAT, :],
                          preferred_element_type=jnp.float32) + dbe[d:d + 1, :])
        h = _celu(jnp.dot(h, dW1[d * LAT:(d + 1) * LAT, :],
                          preferred_element_type=jnp.float32) + db1[d:d + 1, :])
        cols.append(jnp.dot(h, dW2[:, d:d + 1],
                            preferred_element_type=jnp.float32))
    out[...] = jnp.concatenate(cols, axis=1) + db2[...]


def _full(shape):
    return pl.BlockSpec(shape, lambda i: (0,) * len(shape))


def _rows(blk, width):
    return pl.BlockSpec((blk, width), lambda i: (i, 0))


def _tc_enc_nodes(V, p, wr, ws):
    (W0, b0), (W1, b1), (W2, b2), (g, beta) = p
    n = N_NODES // BLK_N
    args = [V, W0, b0.reshape(1, -1), W1, b1.reshape(1, -1), W2,
            b2.reshape(1, -1), g.reshape(1, -1), beta.reshape(1, -1), wr, ws]
    specs = [_rows(BLK_N, LAT)] + [_full(a.shape) for a in args[1:]]
    return pl.pallas_call(
        _enc_nodes_body,
        grid=(n,),
        in_specs=specs,
        out_specs=[_rows(BLK_N, LAT)] * 3,
        out_shape=[jax.ShapeDtypeStruct((N_NODES, LAT), jnp.float32)] * 3,
    )(*args)


def _tc_enc_edges(E, p):
    (W0, b0), (W1, b1), (W2, b2), (g, beta) = p
    n = N_EDGES // BLK_E
    args = [E, W0, b0.reshape(1, -1), W1, b1.reshape(1, -1), W2,
            b2.reshape(1, -1), g.reshape(1, -1), beta.reshape(1, -1)]
    specs = [_rows(BLK_E, E.shape[1])] + [_full(a.shape) for a in args[1:]]
    return pl.pallas_call(
        _enc_edges_body,
        grid=(n,),
        in_specs=specs,
        out_specs=_rows(BLK_E, LAT),
        out_shape=jax.ShapeDtypeStruct((N_EDGES, LAT), jnp.float32),
    )(*args)


def _tc_edge_tail(El, G1, G2, p):
    (W0, b0), (W1, b1), (W2, b2), (g, beta) = p
    W0a = W0[:LAT, :]
    n = N_EDGES // BLK_E
    args = [El, G1, G2, W0a, b0.reshape(1, -1), W1, b1.reshape(1, -1), W2,
            b2.reshape(1, -1), g.reshape(1, -1), beta.reshape(1, -1)]
    specs = [_rows(BLK_E, LAT)] * 3 + [_full(a.shape) for a in args[3:]]
    return pl.pallas_call(
        _edge_tail_body,
        grid=(n,),
        in_specs=specs,
        out_specs=[_rows(BLK_E, LAT)] * 2,
        out_shape=[jax.ShapeDtypeStruct((N_EDGES, LAT), jnp.float32)] * 2,
    )(*args)


def _tc_node_tail(Vl, A, B, p, wr=None, ws=None):
    (W0, b0), (W1, b1), (W2, b2), (g, beta) = p
    W0a, W0b = W0[:LAT, :], W0[LAT:, :]
    n = N_NODES // BLK_N
    args = [Vl, A, B, W0a, W0b, b0.reshape(1, -1), W1, b1.reshape(1, -1), W2,
            b2.reshape(1, -1), g.reshape(1, -1), beta.reshape(1, -1)]
    specs = [_rows(BLK_N, LAT)] * 3 + [_full(a.shape) for a in args[3:]]
    if wr is not None:
        args += [wr, ws]
        specs += [_full(wr.shape), _full(ws.shape)]
        return pl.pallas_call(
            _node_tail_body,
            grid=(n,),
            in_specs=specs,
            out_specs=[_rows(BLK_N, LAT)] * 3,
            out_shape=[jax.ShapeDtypeStruct((N_NODES, LAT), jnp.float32)] * 3,
        )(*args)
    return pl.pallas_call(
        _node_tail_last_body,
        grid=(n,),
        in_specs=specs,
        out_specs=_rows(BLK_N, LAT),
        out_shape=jax.ShapeDtypeStruct((N_NODES, LAT), jnp.float32),
    )(*args)


def _tc_theta(theta2d, p, dW0t, db0):
    (W0, b0), (W1, b1), (W2, b2), (g, beta) = p
    args = [theta2d, W0, b0.reshape(1, -1), W1, b1.reshape(1, -1), W2,
            b2.reshape(1, -1), g.reshape(1, -1), beta.reshape(1, -1), dW0t, db0]
    return pl.pallas_call(
        _theta_body,
        grid=(1,),
        in_specs=[_full(a.shape) for a in args],
        out_specs=_full((3, LAT)),
        out_shape=jax.ShapeDtypeStruct((3, LAT), jnp.float32),
    )(*args)


def _tc_final(Vl, Inc0, A1, A2, mask, g_f, beta_f, dW0z, dbe, dW1, db1, dW2,
              db2):
    n = N_NODES // BLK_N
    args = [Vl, Inc0, A1, A2, mask, g_f.reshape(1, -1), beta_f.reshape(1, -1),
            dW0z, dbe, dW1, db1, dW2, db2]
    specs = ([_rows(BLK_N, LAT)] * 4 + [_rows(BLK_N, 1)]
             + [_full(a.shape) for a in args[5:]])
    return pl.pallas_call(
        _final_body,
        grid=(n,),
        in_specs=specs,
        out_specs=_rows(BLK_N, 3),
        out_shape=jax.ShapeDtypeStruct((N_NODES, 3), jnp.float32),
    )(*args)


_MESH = plsc.VectorSubcoreMesh(core_axis_name="c", subcore_axis_name="s")

_G_BASE = N_CHUNKS // NW
_G_EXTRA = N_CHUNKS - _G_BASE * NW
_S_BASE = N_CHUNKS // NS
_S_EXTRA = N_CHUNKS - _S_BASE * NS
_HALF_N = N_NODES // NC
_TAB_ROWS = _HALF_N + 8
_INIT_R = 312
_OUT_R = 312


def _gather_body(pr_hbm, ps_hbm, recv_hbm, send_hbm, g1_hbm, g2_hbm,
                 idxr0, idxs0, idxr1, idxs1, bufr0, bufs0, bufr1, bufs1,
                 semg0, semg1, semw0, semw1):
    c = lax.axis_index("c")
    s = lax.axis_index("s")
    w = s * NC + c
    nw = jnp.where(w < _G_EXTRA, _G_BASE + 1, _G_BASE)
    start = _G_BASE * w + jnp.minimum(w, _G_EXTRA)

    def load_idx(i, ir, is_):
        off = (start + i) * CH
        pltpu.sync_copy(recv_hbm.at[pl.ds(off, CH)], ir)
        pltpu.sync_copy(send_hbm.at[pl.ds(off, CH)], is_)

    def body(p, carry):
        i0 = 2 * p
        i1 = i0 + 1
        load_idx(i0, idxr0, idxs0)
        dg0a = pltpu.async_copy(pr_hbm.at[idxr0], bufr0, semg0)
        dg0b = pltpu.async_copy(ps_hbm.at[idxs0], bufs0, semg0)

        @pl.when(i1 < nw)
        def _():
            load_idx(i1, idxr1, idxs1)
            dg1a = pltpu.async_copy(pr_hbm.at[idxr1], bufr1, semg1)
            dg1b = pltpu.async_copy(ps_hbm.at[idxs1], bufs1, semg1)
            dg0a.wait()
            dg0b.wait()
            off0 = (i0 + start) * CH
            dw0a = pltpu.async_copy(bufr0, g1_hbm.at[pl.ds(off0, CH)], semw0)
            dw0b = pltpu.async_copy(bufs0, g2_hbm.at[pl.ds(off0, CH)], semw0)
            dg1a.wait()
            dg1b.wait()
            off1 = (i1 + start) * CH
            dw1a = pltpu.async_copy(bufr1, g1_hbm.at[pl.ds(off1, CH)], semw1)
            dw1b = pltpu.async_copy(bufs1, g2_hbm.at[pl.ds(off1, CH)], semw1)
            dw0a.wait()
            dw0b.wait()
            dw1a.wait()
            dw1b.wait()

        @pl.when(i1 >= nw)
        def _():
            dg0a.wait()
            dg0b.wait()
            off0 = (i0 + start) * CH
            dw0a = pltpu.async_copy(bufr0, g1_hbm.at[pl.ds(off0, CH)], semw0)
            dw0b = pltpu.async_copy(bufs0, g2_hbm.at[pl.ds(off0, CH)], semw0)
            dw0a.wait()
            dw0b.wait()

        return carry

    lax.fori_loop(0, (nw + 1) // 2, body, 0)


@functools.partial(
    pl.kernel,
    out_type=[jax.ShapeDtypeStruct((N_EDGES, LAT), jnp.float32)] * 2,
    mesh=_MESH,
    scratch_types=[
        pltpu.VMEM((CH,), jnp.int32),
        pltpu.VMEM((CH,), jnp.int32),
        pltpu.VMEM((CH,), jnp.int32),
        pltpu.VMEM((CH,), jnp.int32),
        pltpu.VMEM((CH, LAT), jnp.float32),
        pltpu.VMEM((CH, LAT), jnp.float32),
        pltpu.VMEM((CH, LAT), jnp.float32),
        pltpu.VMEM((CH, LAT), jnp.float32),
        pltpu.SemaphoreType.DMA,
        pltpu.SemaphoreType.DMA,
        pltpu.SemaphoreType.DMA,
        pltpu.SemaphoreType.DMA,
    ],
)
def _sc_gather(pr_hbm, ps_hbm, recv_hbm, send_hbm, g1_hbm, g2_hbm,
               idxr0, idxs0, idxr1, idxs1, bufr0, bufs0, bufr1, bufs1,
               semg0, semg1, semw0, semw1):
    _gather_body(pr_hbm, ps_hbm, recv_hbm, send_hbm, g1_hbm, g2_hbm,
                 idxr0, idxs0, idxr1, idxs1, bufr0, bufs0, bufr1, bufs1,
                 semg0, semg1, semw0, semw1)


def _tab_init_all(zeros_hbm, table, s):
    for t in range(NS):
        @pl.when(s == t)
        def _():
            r0 = t * _INIT_R
            nr = _TAB_ROWS - 15 * _INIT_R if t == 15 else _INIT_R
            pltpu.sync_copy(zeros_hbm.at[pl.ds(r0, nr)], table.at[pl.ds(r0, nr)])


def _tab_writeout(table, out_hbm, c, s):
    for t in range(NS):
        @pl.when(s == t)
        def _():
            r0 = t * _OUT_R
            nr = _HALF_N - 15 * _OUT_R if t == 15 else _OUT_R
            pltpu.sync_copy(table.at[pl.ds(r0, nr)],
                            out_hbm.at[pl.ds(c * _HALF_N + r0, nr)])


def _clamp_whole(idx_ref, c):
    base = c * _HALF_N
    for j in range(CH // 16):
        v = idx_ref[pl.ds(j * 16, 16)]
        local = v - base
        ok = (local >= 0) & (local < _HALF_N)
        idx_ref[pl.ds(j * 16, 16)] = jnp.where(ok, local, _HALF_N)


def _scatter_pipelined(m_hbm, idx_hbms, tabs, slots, semS, c, start, n):
    def load(i, slot):
        buf, idxs, semL = slot
        off = (start + i) * CH
        pltpu.async_copy(m_hbm.at[pl.ds(off, CH)], buf, semL)
        for ih, ir in zip(idx_hbms, idxs):
            pltpu.async_copy(ih.at[pl.ds(off, CH)], ir, semL)

    def drain_load(slot):
        buf, idxs, semL = slot
        pltpu.make_async_copy(m_hbm.at[pl.ds(0, CH)], buf, semL).wait()
        for ih, ir in zip(idx_hbms, idxs):
            pltpu.make_async_copy(ih.at[pl.ds(0, CH)], ir, semL).wait()

    def scatter(slot):
        buf, idxs, _ = slot
        ds = []
        for tab, ir in zip(tabs, idxs):
            _clamp_whole(ir, c)
            ds.append(pltpu.async_copy(buf, tab.at[ir], semS, add=True))
        for d in ds:
            d.wait()

    load(0, slots[0])

    def body(p, carry):
        i1 = 2 * p + 1
        drain_load(slots[0])

        @pl.when(i1 < n)
        def _():
            load(i1, slots[1])

        scatter(slots[0])

        @pl.when(i1 < n)
        def _():
            drain_load(slots[1])

            @pl.when(i1 + 1 < n)
            def _():
                load(i1 + 1, slots[0])

            scatter(slots[1])

        return carry

    lax.fori_loop(0, (n + 1) // 2, body, 0)


def _scatter2_body(m_hbm, recv_hbm, send_hbm, zeros_hbm, a_hbm, b_hbm,
                   idxr0, idxs0, idxr1, idxs1, buf0, buf1, tabA, tabB,
                   semL0, semL1, semS):
    c = lax.axis_index("c")
    s = lax.axis_index("s")
    _tab_init_all(zeros_hbm, tabA, s)
    _tab_init_all(zeros_hbm, tabB, s)
    n = jnp.where(s < _S_EXTRA, _S_BASE + 1, _S_BASE)
    start = _S_BASE * s + jnp.minimum(s, _S_EXTRA)
    plsc.subcore_barrier()
    _scatter_pipelined(m_hbm, [recv_hbm, send_hbm], [tabA, tabB],
                       [(buf0, [idxr0, idxs0], semL0),
                        (buf1, [idxr1, idxs1], semL1)],
                       semS, c, start, n)
    plsc.subcore_barrier()
    _tab_writeout(tabA, a_hbm, c, s)
    _tab_writeout(tabB, b_hbm, c, s)


@functools.partial(
    pl.kernel,
    out_type=[jax.ShapeDtypeStruct((N_NODES, LAT), jnp.float32)] * 2,
    mesh=_MESH,
    scratch_types=[
        pltpu.VMEM((CH,), jnp.int32),
        pltpu.VMEM((CH,), jnp.int32),
        pltpu.VMEM((CH,), jnp.int32),
        pltpu.VMEM((CH,), jnp.int32),
        pltpu.VMEM((CH, LAT), jnp.float32),
        pltpu.VMEM((CH, LAT), jnp.float32),
        pltpu.VMEM_SHARED((_TAB_ROWS, LAT), jnp.float32),
        pltpu.VMEM_SHARED((_TAB_ROWS, LAT), jnp.float32),
        pltpu.SemaphoreType.DMA,
        pltpu.SemaphoreType.DMA,
        pltpu.SemaphoreType.DMA,
    ],
)
def _sc_scatter2(m_hbm, recv_hbm, send_hbm, zeros_hbm, a_hbm, b_hbm,
                 idxr0, idxs0, idxr1, idxs1, buf0, buf1, tabA, tabB,
                 semL0, semL1, semS):
    _scatter2_body(m_hbm, recv_hbm, send_hbm, zeros_hbm, a_hbm, b_hbm,
                   idxr0, idxs0, idxr1, idxs1, buf0, buf1, tabA, tabB,
                   semL0, semL1, semS)


def _scatter1_body(m_hbm, recv_hbm, zeros_hbm, a_hbm,
                   idxr0, idxr1, buf0, buf1, tabA, semL0, semL1, semS):
    c = lax.axis_index("c")
    s = lax.axis_index("s")
    _tab_init_all(zeros_hbm, tabA, s)
    n = jnp.where(s < _S_EXTRA, _S_BASE + 1, _S_BASE)
    start = _S_BASE * s + jnp.minimum(s, _S_EXTRA)
    plsc.subcore_barrier()
    _scatter_pipelined(m_hbm, [recv_hbm], [tabA],
                       [(buf0, [idxr0], semL0), (buf1, [idxr1], semL1)],
                       semS, c, start, n)
    plsc.subcore_barrier()
    _tab_writeout(tabA, a_hbm, c, s)


@functools.partial(
    pl.kernel,
    out_type=jax.ShapeDtypeStruct((N_NODES, LAT), jnp.float32),
    mesh=_MESH,
    scratch_types=[
        pltpu.VMEM((CH,), jnp.int32),
        pltpu.VMEM((CH,), jnp.int32),
        pltpu.VMEM((CH, LAT), jnp.float32),
        pltpu.VMEM((CH, LAT), jnp.float32),
        pltpu.VMEM_SHARED((_TAB_ROWS, LAT), jnp.float32),
        pltpu.SemaphoreType.DMA,
        pltpu.SemaphoreType.DMA,
        pltpu.SemaphoreType.DMA,
    ],
)
def _sc_scatter1(m_hbm, recv_hbm, zeros_hbm, a_hbm,
                 idxr0, idxr1, buf0, buf1, tabA, semL0, semL1, semS):
    _scatter1_body(m_hbm, recv_hbm, zeros_hbm, a_hbm,
                   idxr0, idxr1, buf0, buf1, tabA, semL0, semL1, semS)


def kernel(V, E, theta, params, senders, receivers, real_node_indices):
    zeros_tab = jnp.zeros((_TAB_ROWS, LAT), jnp.float32)
    mask = real_node_indices.astype(jnp.float32).reshape(N_NODES, 1)
    theta2d = theta.reshape(1, -1)
    mp = params['mp']
    wr = [blk['edge'][0][0][LAT:2 * LAT, :] for blk in mp]
    ws = [blk['edge'][0][0][2 * LAT:, :] for blk in mp]

    Vl, Pr, Ps = _tc_enc_nodes(V, params['node_enc'], wr[0], ws[0])
    El = _tc_enc_edges(E, params['edge_enc'])
    Inc0 = _sc_scatter1(El, receivers, zeros_tab)

    K = len(mp)
    A_list = []
    for k in range(K):
        G1, G2 = _sc_gather(Pr, Ps, receivers, senders)
        M, El = _tc_edge_tail(El, G1, G2, mp[k]['edge'])
        A, B = _sc_scatter2(M, receivers, senders, zeros_tab)
        A_list.append(A)
        if k + 1 < K:
            Vl, Pr, Ps = _tc_node_tail(Vl, A, B, mp[k]['node'],
                                       wr[k + 1], ws[k + 1])
        else:
            Vl = _tc_node_tail(Vl, A, B, mp[k]['node'])

    dec = params['dec']
    dW0t = jnp.concatenate([dec[d][0][0][:LAT, :] for d in range(3)], axis=0)
    dW0z = jnp.concatenate([dec[d][0][0][LAT:, :] for d in range(3)], axis=0)
    db0 = jnp.stack([dec[d][0][1] for d in range(3)])
    dW1 = jnp.concatenate([dec[d][1][0] for d in range(3)], axis=0)
    db1 = jnp.stack([dec[d][1][1] for d in range(3)])
    dW2 = jnp.concatenate([dec[d][2][0] for d in range(3)], axis=1)
    db2 = jnp.stack([dec[d][2][1] for d in range(3)]).reshape(1, 3)

    dbe = _tc_theta(theta2d, params['theta_enc'], dW0t, db0)
    g_f, beta_f = params['final_ln']
    return _tc_final(Vl, Inc0, A_list[0], A_list[1], mask, g_f, beta_f,
                     dW0z, dbe, dW1, db1, dW2, db2)


# speedup vs baseline: 4.1753x; 1.4081x over previous
import functools

import jax
import jax.numpy as jnp
from jax import lax
from jax.experimental import pallas as pl
from jax.experimental.pallas import tpu as pltpu
from jax.experimental.pallas import tpu_sc as plsc

N_NODES = 10000
N_EDGES = 160000
LAT = 128
CH = 128
N_CHUNKS = N_EDGES // CH
NC, NS = 2, 16
NW = NC * NS
BLK_N = 1000
BLK_E = 1000
HALF = LAT // 2


def _celu(x):
    return jnp.where(x > 0, x, jnp.exp(jnp.minimum(x, 0.0)) - 1.0)


def _ln(x, g, beta):
    mu = jnp.mean(x, axis=-1, keepdims=True)
    d = x - mu
    var = jnp.mean(d * d, axis=-1, keepdims=True)
    return d * lax.rsqrt(var + 1e-6) * g + beta


def _mlp3(x, W0, b0, W1, b1, W2, b2, g, beta):
    h = _celu(jnp.dot(x, W0, preferred_element_type=jnp.float32) + b0)
    h = _celu(jnp.dot(h, W1, preferred_element_type=jnp.float32) + b1)
    h = jnp.dot(h, W2, preferred_element_type=jnp.float32) + b2
    return _ln(h, g, beta)


def _enc_nodes_body(v, W0, b0, W1, b1, W2, b2, g, beta, wr, ws, vl_o, pr_o, ps_o):
    vl = _mlp3(v[...], W0[...], b0[...], W1[...], b1[...], W2[...], b2[...],
               g[...], beta[...])
    vl_o[...] = vl
    pr_o[...] = jnp.dot(vl, wr[...], preferred_element_type=jnp.float32)
    ps_o[...] = jnp.dot(vl, ws[...], preferred_element_type=jnp.float32)


def _edge_enc_tail_body(e, eW0, eb0, eW1, eb1, eW2, eb2, eg, ebeta,
                        g1, g2, W0a, b0, W1, b1, W2, b2, g, beta, m_o, eln_o):
    el = _mlp3(e[...], eW0[...], eb0[...], eW1[...], eb1[...], eW2[...],
               eb2[...], eg[...], ebeta[...])
    h = _celu(jnp.dot(el, W0a[...], preferred_element_type=jnp.float32)
              + g1[...] + g2[...] + b0[...])
    h = _celu(jnp.dot(h, W1[...], preferred_element_type=jnp.float32) + b1[...])
    h = jnp.dot(h, W2[...], preferred_element_type=jnp.float32) + b2[...]
    m = _ln(h, g[...], beta[...])
    m_o[...] = m
    eln_o[...] = el + m


def _edge_tail_last_body(el, g1, g2, W0a, b0, W1, b1, W2, b2, g, beta, m_o):
    x = el[...]
    h = _celu(jnp.dot(x, W0a[...], preferred_element_type=jnp.float32)
              + g1[...] + g2[...] + b0[...])
    h = _celu(jnp.dot(h, W1[...], preferred_element_type=jnp.float32) + b1[...])
    h = jnp.dot(h, W2[...], preferred_element_type=jnp.float32) + b2[...]
    m_o[...] = _ln(h, g[...], beta[...])


def _node_tail_body(vl, a, b, W0a, W0b, b0, W1, b1, W2, b2, g, beta, wr, ws,
                    vln_o, pr_o, ps_o):
    x = vl[...]
    s = a[...] - b[...]
    h = _celu(jnp.dot(x, W0a[...], preferred_element_type=jnp.float32)
              + jnp.dot(s, W0b[...], preferred_element_type=jnp.float32)
              + b0[...])
    h = _celu(jnp.dot(h, W1[...], preferred_element_type=jnp.float32) + b1[...])
    h = jnp.dot(h, W2[...], preferred_element_type=jnp.float32) + b2[...]
    vln = x + _ln(h, g[...], beta[...])
    vln_o[...] = vln
    if pr_o is not None:
        pr_o[...] = jnp.dot(vln, wr[...], preferred_element_type=jnp.float32)
        ps_o[...] = jnp.dot(vln, ws[...], preferred_element_type=jnp.float32)


def _node_final_body(vl, a, b, inca, incb, mask, W0a, W0b, b0, W1, b1, W2, b2,
                     g, beta, g_f, beta_f, dW0z, dbe, dW1, db1, dW2, db2, out):
    x = vl[...]
    s = a[...] - b[...]
    h = _celu(jnp.dot(x, W0a[...], preferred_element_type=jnp.float32)
              + jnp.dot(s, W0b[...], preferred_element_type=jnp.float32)
              + b0[...])
    h = _celu(jnp.dot(h, W1[...], preferred_element_type=jnp.float32) + b1[...])
    h = jnp.dot(h, W2[...], preferred_element_type=jnp.float32) + b2[...]
    vln = x + _ln(h, g[...], beta[...])
    inc = inca[...] + incb[...] + a[...]
    m = mask[...]
    z = jnp.concatenate([vln * m, inc * m], axis=1)
    zl = _ln(z, g_f[...], beta_f[...])
    cols = []
    for d in range(3):
        hd = _celu(jnp.dot(zl, dW0z[d * 2 * LAT:(d + 1) * 2 * LAT, :],
                           preferred_element_type=jnp.float32) + dbe[d:d + 1, :])
        hd = _celu(jnp.dot(hd, dW1[d * LAT:(d + 1) * LAT, :],
                           preferred_element_type=jnp.float32) + db1[d:d + 1, :])
        cols.append(jnp.dot(hd, dW2[:, d:d + 1],
                            preferred_element_type=jnp.float32))
    out[...] = jnp.concatenate(cols, axis=1) + db2[...]


def _theta_body(t, W0, b0, W1, b1, W2, b2, g, beta, dW0t, db0, out):
    h = _celu(jnp.dot(t[...], W0[...], preferred_element_type=jnp.float32) + b0[...])
    h = _celu(jnp.dot(h, W1[...], preferred_element_type=jnp.float32) + b1[...])
    h = jnp.dot(h, W2[...], preferred_element_type=jnp.float32) + b2[...]
    zt = _ln(h, g[...], beta[...])
    rows = []
    for d in range(3):
        wd = dW0t[d * LAT:(d + 1) * LAT, :]
        rows.append(jnp.dot(zt, wd, preferred_element_type=jnp.float32)
                    + db0[d:d + 1, :])
    out[...] = jnp.concatenate(rows, axis=0)


def _full(shape):
    return pl.BlockSpec(shape, lambda i: (0,) * len(shape))


def _rows(blk, width):
    return pl.BlockSpec((blk, width), lambda i: (i, 0))


def _tc_enc_nodes(V, p, wr, ws):
    (W0, b0), (W1, b1), (W2, b2), (g, beta) = p
    n = N_NODES // BLK_N
    args = [V, W0, b0.reshape(1, -1), W1, b1.reshape(1, -1), W2,
            b2.reshape(1, -1), g.reshape(1, -1), beta.reshape(1, -1), wr, ws]
    specs = [_rows(BLK_N, LAT)] + [_full(a.shape) for a in args[1:]]
    return pl.pallas_call(
        _enc_nodes_body,
        grid=(n,),
        in_specs=specs,
        out_specs=[_rows(BLK_N, LAT)] * 3,
        out_shape=[jax.ShapeDtypeStruct((N_NODES, LAT), jnp.float32)] * 3,
    )(*args)


def _tc_edge_enc_tail(E, G1, G2, enc_p, p):
    (eW0, eb0), (eW1, eb1), (eW2, eb2), (eg, ebeta) = enc_p
    (W0, b0), (W1, b1), (W2, b2), (g, beta) = p
    W0a = W0[:LAT, :]
    n = N_EDGES // BLK_E
    args = [E, eW0, eb0.reshape(1, -1), eW1, eb1.reshape(1, -1), eW2,
            eb2.reshape(1, -1), eg.reshape(1, -1), ebeta.reshape(1, -1),
            G1, G2, W0a, b0.reshape(1, -1), W1, b1.reshape(1, -1), W2,
            b2.reshape(1, -1), g.reshape(1, -1), beta.reshape(1, -1)]
    specs = ([_rows(BLK_E, E.shape[1])]
             + [_full(a.shape) for a in args[1:9]]
             + [_rows(BLK_E, LAT)] * 2
             + [_full(a.shape) for a in args[11:]])
    return pl.pallas_call(
        _edge_enc_tail_body,
        grid=(n,),
        in_specs=specs,
        out_specs=[_rows(BLK_E, LAT)] * 2,
        out_shape=[jax.ShapeDtypeStruct((N_EDGES, LAT), jnp.float32)] * 2,
    )(*args)


def _tc_edge_tail_last(El, G1, G2, p):
    (W0, b0), (W1, b1), (W2, b2), (g, beta) = p
    W0a = W0[:LAT, :]
    n = N_EDGES // BLK_E
    args = [El, G1, G2, W0a, b0.reshape(1, -1), W1, b1.reshape(1, -1), W2,
            b2.reshape(1, -1), g.reshape(1, -1), beta.reshape(1, -1)]
    specs = [_rows(BLK_E, LAT)] * 3 + [_full(a.shape) for a in args[3:]]
    return pl.pallas_call(
        _edge_tail_last_body,
        grid=(n,),
        in_specs=specs,
        out_specs=_rows(BLK_E, LAT),
        out_shape=jax.ShapeDtypeStruct((N_EDGES, LAT), jnp.float32),
    )(*args)


def _tc_node_tail(Vl, A, B, p, wr, ws):
    (W0, b0), (W1, b1), (W2, b2), (g, beta) = p
    W0a, W0b = W0[:LAT, :], W0[LAT:, :]
    n = N_NODES // BLK_N
    args = [Vl, A, B, W0a, W0b, b0.reshape(1, -1), W1, b1.reshape(1, -1), W2,
            b2.reshape(1, -1), g.reshape(1, -1), beta.reshape(1, -1), wr, ws]
    specs = ([_rows(BLK_N, LAT)] * 3
             + [_full(a.shape) for a in args[3:]])
    return pl.pallas_call(
        _node_tail_body,
        grid=(n,),
        in_specs=specs,
        out_specs=[_rows(BLK_N, LAT)] * 3,
        out_shape=[jax.ShapeDtypeStruct((N_NODES, LAT), jnp.float32)] * 3,
    )(*args)


def _tc_node_final(Vl, A, B, IncA, IncB, mask, p, g_f, beta_f,
                   dW0z, dbe, dW1, db1, dW2, db2):
    (W0, b0), (W1, b1), (W2, b2), (g, beta) = p
    W0a, W0b = W0[:LAT, :], W0[LAT:, :]
    n = N_NODES // BLK_N
    args = [Vl, A, B, IncA, IncB, mask, W0a, W0b, b0.reshape(1, -1),
            W1, b1.reshape(1, -1), W2, b2.reshape(1, -1),
            g.reshape(1, -1), beta.reshape(1, -1),
            g_f.reshape(1, -1), beta_f.reshape(1, -1),
            dW0z, dbe, dW1, db1, dW2, db2]
    specs = ([_rows(BLK_N, LAT)] * 5 + [_rows(BLK_N, 1)]
             + [_full(a.shape) for a in args[6:]])
    return pl.pallas_call(
        _node_final_body,
        grid=(n,),
        in_specs=specs,
        out_specs=_rows(BLK_N, 3),
        out_shape=jax.ShapeDtypeStruct((N_NODES, 3), jnp.float32),
    )(*args)


def _tc_theta(theta2d, p, dW0t, db0):
    (W0, b0), (W1, b1), (W2, b2), (g, beta) = p
    args = [theta2d, W0, b0.reshape(1, -1), W1, b1.reshape(1, -1), W2,
            b2.reshape(1, -1), g.reshape(1, -1), beta.reshape(1, -1), dW0t, db0]
    return pl.pallas_call(
        _theta_body,
        grid=(1,),
        in_specs=[_full(a.shape) for a in args],
        out_specs=_full((3, LAT)),
        out_shape=jax.ShapeDtypeStruct((3, LAT), jnp.float32),
    )(*args)


_MESH = plsc.VectorSubcoreMesh(core_axis_name="c", subcore_axis_name="s")

_G_BASE = N_CHUNKS // NW
_G_EXTRA = N_CHUNKS - _G_BASE * NW
_S_BASE = N_CHUNKS // NS
_S_EXTRA = N_CHUNKS - _S_BASE * NS
_TAB_SPAN = 624


def _gather_body(pr_hbm, ps_hbm, recv_hbm, send_hbm, g1_hbm, g2_hbm,
                 idxr0, idxs0, idxr1, idxs1, bufr0, bufs0, bufr1, bufs1,
                 semg0, semg1, semw0, semw1):
    c = lax.axis_index("c")
    s = lax.axis_index("s")
    w = s * NC + c
    nw = jnp.where(w < _G_EXTRA, _G_BASE + 1, _G_BASE)
    start = _G_BASE * w + jnp.minimum(w, _G_EXTRA)

    def load_idx(i, ir, is_):
        off = (start + i) * CH
        pltpu.sync_copy(recv_hbm.at[pl.ds(off, CH)], ir)
        pltpu.sync_copy(send_hbm.at[pl.ds(off, CH)], is_)

    def body(p, carry):
        i0 = 2 * p
        i1 = i0 + 1
        load_idx(i0, idxr0, idxs0)
        dg0a = pltpu.async_copy(pr_hbm.at[idxr0], bufr0, semg0)
        dg0b = pltpu.async_copy(ps_hbm.at[idxs0], bufs0, semg0)

        @pl.when(i1 < nw)
        def _():
            load_idx(i1, idxr1, idxs1)
            dg1a = pltpu.async_copy(pr_hbm.at[idxr1], bufr1, semg1)
            dg1b = pltpu.async_copy(ps_hbm.at[idxs1], bufs1, semg1)
            dg0a.wait()
            dg0b.wait()
            off0 = (i0 + start) * CH
            dw0a = pltpu.async_copy(bufr0, g1_hbm.at[pl.ds(off0, CH)], semw0)
            dw0b = pltpu.async_copy(bufs0, g2_hbm.at[pl.ds(off0, CH)], semw0)
            dg1a.wait()
            dg1b.wait()
            off1 = (i1 + start) * CH
            dw1a = pltpu.async_copy(bufr1, g1_hbm.at[pl.ds(off1, CH)], semw1)
            dw1b = pltpu.async_copy(bufs1, g2_hbm.at[pl.ds(off1, CH)], semw1)
            dw0a.wait()
            dw0b.wait()
            dw1a.wait()
            dw1b.wait()

        @pl.when(i1 >= nw)
        def _():
            dg0a.wait()
            dg0b.wait()
            off0 = (i0 + start) * CH
            dw0a = pltpu.async_copy(bufr0, g1_hbm.at[pl.ds(off0, CH)], semw0)
            dw0b = pltpu.async_copy(bufs0, g2_hbm.at[pl.ds(off0, CH)], semw0)
            dw0a.wait()
            dw0b.wait()

        return carry

    lax.fori_loop(0, (nw + 1) // 2, body, 0)


@functools.partial(
    pl.kernel,
    out_type=[jax.ShapeDtypeStruct((N_EDGES, LAT), jnp.float32)] * 2,
    mesh=_MESH,
    scratch_types=[
        pltpu.VMEM((CH,), jnp.int32),
        pltpu.VMEM((CH,), jnp.int32),
        pltpu.VMEM((CH,), jnp.int32),
        pltpu.VMEM((CH,), jnp.int32),
        pltpu.VMEM((CH, LAT), jnp.float32),
        pltpu.VMEM((CH, LAT), jnp.float32),
        pltpu.VMEM((CH, LAT), jnp.float32),
        pltpu.VMEM((CH, LAT), jnp.float32),
        pltpu.SemaphoreType.DMA,
        pltpu.SemaphoreType.DMA,
        pltpu.SemaphoreType.DMA,
        pltpu.SemaphoreType.DMA,
    ],
)
def _sc_gather(pr_hbm, ps_hbm, recv_hbm, send_hbm, g1_hbm, g2_hbm,
               idxr0, idxs0, idxr1, idxs1, bufr0, bufs0, bufr1, bufs1,
               semg0, semg1, semw0, semw1):
    _gather_body(pr_hbm, ps_hbm, recv_hbm, send_hbm, g1_hbm, g2_hbm,
                 idxr0, idxs0, idxr1, idxs1, bufr0, bufs0, bufr1, bufs1,
                 semg0, semg1, semw0, semw1)


def _tab_init_all(zeros_hbm, table, s):
    r0 = s * _TAB_SPAN
    pltpu.sync_copy(zeros_hbm.at[pl.ds(r0, _TAB_SPAN)],
                    table.at[pl.ds(r0, _TAB_SPAN)])

    @pl.when(s == NS - 1)
    def _():
        tail = N_NODES - NS * _TAB_SPAN
        pltpu.sync_copy(zeros_hbm.at[pl.ds(NS * _TAB_SPAN, tail)],
                        table.at[pl.ds(NS * _TAB_SPAN, tail)])


def _tab_writeout_full(table, out_hbm, s):
    r0 = s * _TAB_SPAN
    pltpu.sync_copy(table.at[pl.ds(r0, _TAB_SPAN)],
                    out_hbm.at[pl.ds(r0, _TAB_SPAN)])

    @pl.when(s == NS - 1)
    def _():
        tail = N_NODES - NS * _TAB_SPAN
        pltpu.sync_copy(table.at[pl.ds(NS * _TAB_SPAN, tail)],
                        out_hbm.at[pl.ds(NS * _TAB_SPAN, tail)])


def _scatter_pipelined(m_hbm, idx_hbm, tab, slots, semS, start, n):
    def load(i, slot):
        buf, ir, semL = slot
        off = (start + i) * CH
        pltpu.async_copy(m_hbm.at[pl.ds(off, CH)], buf, semL)
        pltpu.async_copy(idx_hbm.at[pl.ds(off, CH)], ir, semL)

    def drain_load(slot):
        buf, ir, semL = slot
        pltpu.make_async_copy(m_hbm.at[pl.ds(0, CH)], buf, semL).wait()
        pltpu.make_async_copy(idx_hbm.at[pl.ds(0, CH)], ir, semL).wait()

    def scatter(slot):
        buf, ir, _ = slot
        pltpu.async_copy(buf, tab.at[ir], semS, add=True).wait()

    load(0, slots[0])

    def body(p, carry):
        i1 = 2 * p + 1
        drain_load(slots[0])

        @pl.when(i1 < n)
        def _():
            load(i1, slots[1])

        scatter(slots[0])

        @pl.when(i1 < n)
        def _():
            drain_load(slots[1])

            @pl.when(i1 + 1 < n)
            def _():
                load(i1 + 1, slots[0])

            scatter(slots[1])

        return carry

    lax.fori_loop(0, (n + 1) // 2, body, 0)


def _scatter2_body(m_hbm, recv_hbm, send_hbm, zeros_hbm, a_hbm, b_hbm,
                   idx0, idx1, buf0, buf1, tab, semL0, semL1, semS):
    c = lax.axis_index("c")
    s = lax.axis_index("s")
    _tab_init_all(zeros_hbm, tab, s)
    n = jnp.where(s < _S_EXTRA, _S_BASE + 1, _S_BASE)
    start = _S_BASE * s + jnp.minimum(s, _S_EXTRA)
    plsc.subcore_barrier()

    @pl.when(c == 0)
    def _():
        _scatter_pipelined(m_hbm, recv_hbm, tab,
                           [(buf0, idx0, semL0), (buf1, idx1, semL1)],
                           semS, start, n)

    @pl.when(c == 1)
    def _():
        _scatter_pipelined(m_hbm, send_hbm, tab,
                           [(buf0, idx0, semL0), (buf1, idx1, semL1)],
                           semS, start, n)

    plsc.subcore_barrier()

    @pl.when(c == 0)
    def _():
        _tab_writeout_full(tab, a_hbm, s)

    @pl.when(c == 1)
    def _():
        _tab_writeout_full(tab, b_hbm, s)


@functools.partial(
    pl.kernel,
    out_type=[jax.ShapeDtypeStruct((N_NODES, LAT), jnp.float32)] * 2,
    mesh=_MESH,
    scratch_types=[
        pltpu.VMEM((CH,), jnp.int32),
        pltpu.VMEM((CH,), jnp.int32),
        pltpu.VMEM((CH, LAT), jnp.float32),
        pltpu.VMEM((CH, LAT), jnp.float32),
        pltpu.VMEM_SHARED((N_NODES, LAT), jnp.float32),
        pltpu.SemaphoreType.DMA,
        pltpu.SemaphoreType.DMA,
        pltpu.SemaphoreType.DMA,
    ],
)
def _sc_scatter2(m_hbm, recv_hbm, send_hbm, zeros_hbm, a_hbm, b_hbm,
                 idx0, idx1, buf0, buf1, tab, semL0, semL1, semS):
    _scatter2_body(m_hbm, recv_hbm, send_hbm, zeros_hbm, a_hbm, b_hbm,
                   idx0, idx1, buf0, buf1, tab, semL0, semL1, semS)


_S1_CHUNKS = N_CHUNKS // NC
_S1_BASE = _S1_CHUNKS // NS
_S1_EXTRA = _S1_CHUNKS - _S1_BASE * NS


def _scatter1_body(m_hbm, recv_hbm, zeros_hbm, a_hbm, b_hbm,
                   idx0, idx1, buf0, buf1, tab, semL0, semL1, semS):
    c = lax.axis_index("c")
    s = lax.axis_index("s")
    _tab_init_all(zeros_hbm, tab, s)
    n = jnp.where(s < _S1_EXTRA, _S1_BASE + 1, _S1_BASE)
    start = c * _S1_CHUNKS + _S1_BASE * s + jnp.minimum(s, _S1_EXTRA)
    plsc.subcore_barrier()
    _scatter_pipelined(m_hbm, recv_hbm, tab,
                       [(buf0, idx0, semL0), (buf1, idx1, semL1)],
                       semS, start, n)
    plsc.subcore_barrier()

    @pl.when(c == 0)
    def _():
        _tab_writeout_full(tab, a_hbm, s)

    @pl.when(c == 1)
    def _():
        _tab_writeout_full(tab, b_hbm, s)


@functools.partial(
    pl.kernel,
    out_type=[jax.ShapeDtypeStruct((N_NODES, LAT), jnp.float32)] * 2,
    mesh=_MESH,
    scratch_types=[
        pltpu.VMEM((CH,), jnp.int32),
        pltpu.VMEM((CH,), jnp.int32),
        pltpu.VMEM((CH, LAT), jnp.float32),
        pltpu.VMEM((CH, LAT), jnp.float32),
        pltpu.VMEM_SHARED((N_NODES, LAT), jnp.float32),
        pltpu.SemaphoreType.DMA,
        pltpu.SemaphoreType.DMA,
        pltpu.SemaphoreType.DMA,
    ],
)
def _sc_scatter1(m_hbm, recv_hbm, zeros_hbm, a_hbm, b_hbm,
                 idx0, idx1, buf0, buf1, tab, semL0, semL1, semS):
    _scatter1_body(m_hbm, recv_hbm, zeros_hbm, a_hbm, b_hbm,
                   idx0, idx1, buf0, buf1, tab, semL0, semL1, semS)


def kernel(V, E, theta, params, senders, receivers, real_node_indices):
    zeros_tab = jnp.zeros((N_NODES, LAT), jnp.float32)
    mask = real_node_indices.astype(jnp.float32).reshape(N_NODES, 1)
    theta2d = theta.reshape(1, -1)
    mp = params['mp']
    wr = [blk['edge'][0][0][LAT:2 * LAT, :] for blk in mp]
    ws = [blk['edge'][0][0][2 * LAT:, :] for blk in mp]

    Vl, Pr, Ps = _tc_enc_nodes(V, params['node_enc'], wr[0], ws[0])
    G1, G2 = _sc_gather(Pr, Ps, receivers, senders)
    M0, El1 = _tc_edge_enc_tail(E, G1, G2, params['edge_enc'], mp[0]['edge'])
    A0, B0 = _sc_scatter2(M0, receivers, senders, zeros_tab)
    IncA, IncB = _sc_scatter1(El1, receivers, zeros_tab)
    Vl, Pr, Ps = _tc_node_tail(Vl, A0, B0, mp[0]['node'], wr[1], ws[1])

    G1, G2 = _sc_gather(Pr, Ps, receivers, senders)
    M1 = _tc_edge_tail_last(El1, G1, G2, mp[1]['edge'])
    A1, B1 = _sc_scatter2(M1, receivers, senders, zeros_tab)

    dec = params['dec']
    dW0t = jnp.concatenate([dec[d][0][0][:LAT, :] for d in range(3)], axis=0)
    dW0z = jnp.concatenate([dec[d][0][0][LAT:, :] for d in range(3)], axis=0)
    db0 = jnp.stack([dec[d][0][1] for d in range(3)])
    dW1 = jnp.concatenate([dec[d][1][0] for d in range(3)], axis=0)
    db1 = jnp.stack([dec[d][1][1] for d in range(3)])
    dW2 = jnp.concatenate([dec[d][2][0] for d in range(3)], axis=1)
    db2 = jnp.stack([dec[d][2][1] for d in range(3)]).reshape(1, 3)

    dbe = _tc_theta(theta2d, params['theta_enc'], dW0t, db0)
    g_f, beta_f = params['final_ln']
    return _tc_node_final(Vl, A1, B1, IncA, IncB, mask, mp[1]['node'],
                          g_f, beta_f, dW0z, dbe, dW1, db1, dW2, db2)


# speedup vs baseline: 4.7548x; 1.1388x over previous
import functools

import jax
import jax.numpy as jnp
from jax import lax
from jax.experimental import pallas as pl
from jax.experimental.pallas import tpu as pltpu
from jax.experimental.pallas import tpu_sc as plsc

N_NODES = 10000
N_EDGES = 160000
LAT = 128
CH = 128
N_CHUNKS = N_EDGES // CH
NC, NS = 2, 16
NW = NC * NS
BLK_N = 2000
BLK_E = 2000
HALF = LAT // 2


def _celu(x):
    return jnp.where(x > 0, x, jnp.exp(jnp.minimum(x, 0.0)) - 1.0)


def _ln(x, g, beta):
    mu = jnp.mean(x, axis=-1, keepdims=True)
    d = x - mu
    var = jnp.mean(d * d, axis=-1, keepdims=True)
    return d * lax.rsqrt(var + 1e-6) * g + beta


def _mlp3(x, W0, b0, W1, b1, W2, b2, g, beta):
    h = _celu(jnp.dot(x, W0, preferred_element_type=jnp.float32) + b0)
    h = _celu(jnp.dot(h, W1, preferred_element_type=jnp.float32) + b1)
    h = jnp.dot(h, W2, preferred_element_type=jnp.float32) + b2
    return _ln(h, g, beta)


def _enc_nodes_body(v, W0, b0, W1, b1, W2, b2, g, beta, wr, ws, vl_o, pr_o, ps_o):
    vl = _mlp3(v[...], W0[...], b0[...], W1[...], b1[...], W2[...], b2[...],
               g[...], beta[...])
    vl_o[...] = vl
    pr_o[...] = jnp.dot(vl, wr[...], preferred_element_type=jnp.float32)
    ps_o[...] = jnp.dot(vl, ws[...], preferred_element_type=jnp.float32)


def _edge_enc_tail_body(e, eW0, eb0, eW1, eb1, eW2, eb2, eg, ebeta,
                        g1, g2, W0a, b0, W1, b1, W2, b2, g, beta, m_o, eln_o):
    el = _mlp3(e[...], eW0[...], eb0[...], eW1[...], eb1[...], eW2[...],
               eb2[...], eg[...], ebeta[...])
    h = _celu(jnp.dot(el, W0a[...], preferred_element_type=jnp.float32)
              + g1[...] + g2[...] + b0[...])
    h = _celu(jnp.dot(h, W1[...], preferred_element_type=jnp.float32) + b1[...])
    h = jnp.dot(h, W2[...], preferred_element_type=jnp.float32) + b2[...]
    m = _ln(h, g[...], beta[...])
    m_o[...] = m
    eln_o[...] = el + m


def _edge_tail_last_body(el, g1, g2, W0a, b0, W1, b1, W2, b2, g, beta, m_o):
    x = el[...]
    h = _celu(jnp.dot(x, W0a[...], preferred_element_type=jnp.float32)
              + g1[...] + g2[...] + b0[...])
    h = _celu(jnp.dot(h, W1[...], preferred_element_type=jnp.float32) + b1[...])
    h = jnp.dot(h, W2[...], preferred_element_type=jnp.float32) + b2[...]
    m_o[...] = _ln(h, g[...], beta[...])


def _node_tail_body(vl, a, b, W0a, W0b, b0, W1, b1, W2, b2, g, beta, wr, ws,
                    vln_o, pr_o, ps_o):
    x = vl[...]
    s = a[...] - b[...]
    h = _celu(jnp.dot(x, W0a[...], preferred_element_type=jnp.float32)
              + jnp.dot(s, W0b[...], preferred_element_type=jnp.float32)
              + b0[...])
    h = _celu(jnp.dot(h, W1[...], preferred_element_type=jnp.float32) + b1[...])
    h = jnp.dot(h, W2[...], preferred_element_type=jnp.float32) + b2[...]
    vln = x + _ln(h, g[...], beta[...])
    vln_o[...] = vln
    if pr_o is not None:
        pr_o[...] = jnp.dot(vln, wr[...], preferred_element_type=jnp.float32)
        ps_o[...] = jnp.dot(vln, ws[...], preferred_element_type=jnp.float32)


def _node_final_body(vl, a, b, inca, incb, mask, W0a, W0b, b0, W1, b1, W2, b2,
                     g, beta, g_f, beta_f, dW0z, dbe, dW1, db1, dW2, db2, out):
    x = vl[...]
    s = a[...] - b[...]
    h = _celu(jnp.dot(x, W0a[...], preferred_element_type=jnp.float32)
              + jnp.dot(s, W0b[...], preferred_element_type=jnp.float32)
              + b0[...])
    h = _celu(jnp.dot(h, W1[...], preferred_element_type=jnp.float32) + b1[...])
    h = jnp.dot(h, W2[...], preferred_element_type=jnp.float32) + b2[...]
    vln = x + _ln(h, g[...], beta[...])
    inc = inca[...] + incb[...] + a[...]
    m = mask[...]
    z = jnp.concatenate([vln * m, inc * m], axis=1)
    zl = _ln(z, g_f[...], beta_f[...])
    cols = []
    for d in range(3):
        hd = _celu(jnp.dot(zl, dW0z[d * 2 * LAT:(d + 1) * 2 * LAT, :],
                           preferred_element_type=jnp.float32) + dbe[d:d + 1, :])
        hd = _celu(jnp.dot(hd, dW1[d * LAT:(d + 1) * LAT, :],
                           preferred_element_type=jnp.float32) + db1[d:d + 1, :])
        cols.append(jnp.dot(hd, dW2[:, d:d + 1],
                            preferred_element_type=jnp.float32))
    out[...] = jnp.concatenate(cols, axis=1) + db2[...]


def _theta_body(t, W0, b0, W1, b1, W2, b2, g, beta, dW0t, db0, out):
    h = _celu(jnp.dot(t[...], W0[...], preferred_element_type=jnp.float32) + b0[...])
    h = _celu(jnp.dot(h, W1[...], preferred_element_type=jnp.float32) + b1[...])
    h = jnp.dot(h, W2[...], preferred_element_type=jnp.float32) + b2[...]
    zt = _ln(h, g[...], beta[...])
    rows = []
    for d in range(3):
        wd = dW0t[d * LAT:(d + 1) * LAT, :]
        rows.append(jnp.dot(zt, wd, preferred_element_type=jnp.float32)
                    + db0[d:d + 1, :])
    out[...] = jnp.concatenate(rows, axis=0)


def _full(shape):
    return pl.BlockSpec(shape, lambda i: (0,) * len(shape))


def _rows(blk, width):
    return pl.BlockSpec((blk, width), lambda i: (i, 0))


def _tc_enc_nodes(V, p, wr, ws):
    (W0, b0), (W1, b1), (W2, b2), (g, beta) = p
    n = N_NODES // BLK_N
    args = [V, W0, b0.reshape(1, -1), W1, b1.reshape(1, -1), W2,
            b2.reshape(1, -1), g.reshape(1, -1), beta.reshape(1, -1), wr, ws]
    specs = [_rows(BLK_N, LAT)] + [_full(a.shape) for a in args[1:]]
    return pl.pallas_call(
        _enc_nodes_body,
        grid=(n,),
        in_specs=specs,
        out_specs=[_rows(BLK_N, LAT)] * 3,
        out_shape=[jax.ShapeDtypeStruct((N_NODES, LAT), jnp.float32)] * 3,
    )(*args)


def _tc_edge_enc_tail(E, G1, G2, enc_p, p):
    (eW0, eb0), (eW1, eb1), (eW2, eb2), (eg, ebeta) = enc_p
    (W0, b0), (W1, b1), (W2, b2), (g, beta) = p
    W0a = W0[:LAT, :]
    n = N_EDGES // BLK_E
    args = [E, eW0, eb0.reshape(1, -1), eW1, eb1.reshape(1, -1), eW2,
            eb2.reshape(1, -1), eg.reshape(1, -1), ebeta.reshape(1, -1),
            G1, G2, W0a, b0.reshape(1, -1), W1, b1.reshape(1, -1), W2,
            b2.reshape(1, -1), g.reshape(1, -1), beta.reshape(1, -1)]
    specs = ([_rows(BLK_E, E.shape[1])]
             + [_full(a.shape) for a in args[1:9]]
             + [_rows(BLK_E, LAT)] * 2
             + [_full(a.shape) for a in args[11:]])
    return pl.pallas_call(
        _edge_enc_tail_body,
        grid=(n,),
        in_specs=specs,
        out_specs=[_rows(BLK_E, LAT)] * 2,
        out_shape=[jax.ShapeDtypeStruct((N_EDGES, LAT), jnp.float32)] * 2,
    )(*args)


def _tc_edge_tail_last(El, G1, G2, p):
    (W0, b0), (W1, b1), (W2, b2), (g, beta) = p
    W0a = W0[:LAT, :]
    n = N_EDGES // BLK_E
    args = [El, G1, G2, W0a, b0.reshape(1, -1), W1, b1.reshape(1, -1), W2,
            b2.reshape(1, -1), g.reshape(1, -1), beta.reshape(1, -1)]
    specs = [_rows(BLK_E, LAT)] * 3 + [_full(a.shape) for a in args[3:]]
    return pl.pallas_call(
        _edge_tail_last_body,
        grid=(n,),
        in_specs=specs,
        out_specs=_rows(BLK_E, LAT),
        out_shape=jax.ShapeDtypeStruct((N_EDGES, LAT), jnp.float32),
    )(*args)


def _tc_node_tail(Vl, A, B, p, wr, ws):
    (W0, b0), (W1, b1), (W2, b2), (g, beta) = p
    W0a, W0b = W0[:LAT, :], W0[LAT:, :]
    n = N_NODES // BLK_N
    args = [Vl, A, B, W0a, W0b, b0.reshape(1, -1), W1, b1.reshape(1, -1), W2,
            b2.reshape(1, -1), g.reshape(1, -1), beta.reshape(1, -1), wr, ws]
    specs = ([_rows(BLK_N, LAT)] * 3
             + [_full(a.shape) for a in args[3:]])
    return pl.pallas_call(
        _node_tail_body,
        grid=(n,),
        in_specs=specs,
        out_specs=[_rows(BLK_N, LAT)] * 3,
        out_shape=[jax.ShapeDtypeStruct((N_NODES, LAT), jnp.float32)] * 3,
    )(*args)


def _tc_node_final(Vl, A, B, IncA, IncB, mask, p, g_f, beta_f,
                   dW0z, dbe, dW1, db1, dW2, db2):
    (W0, b0), (W1, b1), (W2, b2), (g, beta) = p
    W0a, W0b = W0[:LAT, :], W0[LAT:, :]
    n = N_NODES // BLK_N
    args = [Vl, A, B, IncA, IncB, mask, W0a, W0b, b0.reshape(1, -1),
            W1, b1.reshape(1, -1), W2, b2.reshape(1, -1),
            g.reshape(1, -1), beta.reshape(1, -1),
            g_f.reshape(1, -1), beta_f.reshape(1, -1),
            dW0z, dbe, dW1, db1, dW2, db2]
    specs = ([_rows(BLK_N, LAT)] * 5 + [_rows(BLK_N, 1)]
             + [_full(a.shape) for a in args[6:]])
    return pl.pallas_call(
        _node_final_body,
        grid=(n,),
        in_specs=specs,
        out_specs=_rows(BLK_N, 3),
        out_shape=jax.ShapeDtypeStruct((N_NODES, 3), jnp.float32),
    )(*args)


def _tc_theta(theta2d, p, dW0t, db0):
    (W0, b0), (W1, b1), (W2, b2), (g, beta) = p
    args = [theta2d, W0, b0.reshape(1, -1), W1, b1.reshape(1, -1), W2,
            b2.reshape(1, -1), g.reshape(1, -1), beta.reshape(1, -1), dW0t, db0]
    return pl.pallas_call(
        _theta_body,
        grid=(1,),
        in_specs=[_full(a.shape) for a in args],
        out_specs=_full((3, LAT)),
        out_shape=jax.ShapeDtypeStruct((3, LAT), jnp.float32),
    )(*args)


_MESH = plsc.VectorSubcoreMesh(core_axis_name="c", subcore_axis_name="s")

_G_BASE = N_CHUNKS // NW
_G_EXTRA = N_CHUNKS - _G_BASE * NW
_S_BASE = N_CHUNKS // NS
_S_EXTRA = N_CHUNKS - _S_BASE * NS
_TAB_SPAN = 624


def _gather_body(pr_hbm, ps_hbm, recv_hbm, send_hbm, g1_hbm, g2_hbm,
                 idxr0, idxs0, idxr1, idxs1, bufr0, bufs0, bufr1, bufs1,
                 semg0, semg1, semw0, semw1):
    c = lax.axis_index("c")
    s = lax.axis_index("s")
    w = s * NC + c
    nw = jnp.where(w < _G_EXTRA, _G_BASE + 1, _G_BASE)
    start = _G_BASE * w + jnp.minimum(w, _G_EXTRA)

    def load_idx(i, ir, is_):
        off = (start + i) * CH
        pltpu.sync_copy(recv_hbm.at[pl.ds(off, CH)], ir)
        pltpu.sync_copy(send_hbm.at[pl.ds(off, CH)], is_)

    def body(p, carry):
        i0 = 2 * p
        i1 = i0 + 1
        load_idx(i0, idxr0, idxs0)
        dg0a = pltpu.async_copy(pr_hbm.at[idxr0], bufr0, semg0)
        dg0b = pltpu.async_copy(ps_hbm.at[idxs0], bufs0, semg0)

        @pl.when(i1 < nw)
        def _():
            load_idx(i1, idxr1, idxs1)
            dg1a = pltpu.async_copy(pr_hbm.at[idxr1], bufr1, semg1)
            dg1b = pltpu.async_copy(ps_hbm.at[idxs1], bufs1, semg1)
            dg0a.wait()
            dg0b.wait()
            off0 = (i0 + start) * CH
            dw0a = pltpu.async_copy(bufr0, g1_hbm.at[pl.ds(off0, CH)], semw0)
            dw0b = pltpu.async_copy(bufs0, g2_hbm.at[pl.ds(off0, CH)], semw0)
            dg1a.wait()
            dg1b.wait()
            off1 = (i1 + start) * CH
            dw1a = pltpu.async_copy(bufr1, g1_hbm.at[pl.ds(off1, CH)], semw1)
            dw1b = pltpu.async_copy(bufs1, g2_hbm.at[pl.ds(off1, CH)], semw1)
            dw0a.wait()
            dw0b.wait()
            dw1a.wait()
            dw1b.wait()

        @pl.when(i1 >= nw)
        def _():
            dg0a.wait()
            dg0b.wait()
            off0 = (i0 + start) * CH
            dw0a = pltpu.async_copy(bufr0, g1_hbm.at[pl.ds(off0, CH)], semw0)
            dw0b = pltpu.async_copy(bufs0, g2_hbm.at[pl.ds(off0, CH)], semw0)
            dw0a.wait()
            dw0b.wait()

        return carry

    lax.fori_loop(0, (nw + 1) // 2, body, 0)


@functools.partial(
    pl.kernel,
    out_type=[jax.ShapeDtypeStruct((N_EDGES, LAT), jnp.float32)] * 2,
    mesh=_MESH,
    scratch_types=[
        pltpu.VMEM((CH,), jnp.int32),
        pltpu.VMEM((CH,), jnp.int32),
        pltpu.VMEM((CH,), jnp.int32),
        pltpu.VMEM((CH,), jnp.int32),
        pltpu.VMEM((CH, LAT), jnp.float32),
        pltpu.VMEM((CH, LAT), jnp.float32),
        pltpu.VMEM((CH, LAT), jnp.float32),
        pltpu.VMEM((CH, LAT), jnp.float32),
        pltpu.SemaphoreType.DMA,
        pltpu.SemaphoreType.DMA,
        pltpu.SemaphoreType.DMA,
        pltpu.SemaphoreType.DMA,
    ],
)
def _sc_gather(pr_hbm, ps_hbm, recv_hbm, send_hbm, g1_hbm, g2_hbm,
               idxr0, idxs0, idxr1, idxs1, bufr0, bufs0, bufr1, bufs1,
               semg0, semg1, semw0, semw1):
    _gather_body(pr_hbm, ps_hbm, recv_hbm, send_hbm, g1_hbm, g2_hbm,
                 idxr0, idxs0, idxr1, idxs1, bufr0, bufs0, bufr1, bufs1,
                 semg0, semg1, semw0, semw1)


def _tab_init_all(zeros_hbm, table, s):
    r0 = s * _TAB_SPAN
    pltpu.sync_copy(zeros_hbm.at[pl.ds(r0, _TAB_SPAN)],
                    table.at[pl.ds(r0, _TAB_SPAN)])

    @pl.when(s == NS - 1)
    def _():
        tail = N_NODES - NS * _TAB_SPAN
        pltpu.sync_copy(zeros_hbm.at[pl.ds(NS * _TAB_SPAN, tail)],
                        table.at[pl.ds(NS * _TAB_SPAN, tail)])


def _tab_writeout_full(table, out_hbm, s):
    r0 = s * _TAB_SPAN
    pltpu.sync_copy(table.at[pl.ds(r0, _TAB_SPAN)],
                    out_hbm.at[pl.ds(r0, _TAB_SPAN)])

    @pl.when(s == NS - 1)
    def _():
        tail = N_NODES - NS * _TAB_SPAN
        pltpu.sync_copy(table.at[pl.ds(NS * _TAB_SPAN, tail)],
                        out_hbm.at[pl.ds(NS * _TAB_SPAN, tail)])


def _scatter_pipelined(m_hbm, idx_hbm, tab, slots, semS, start, n):
    def load(i, slot):
        buf, ir, semL = slot
        off = (start + i) * CH
        pltpu.async_copy(m_hbm.at[pl.ds(off, CH)], buf, semL)
        pltpu.async_copy(idx_hbm.at[pl.ds(off, CH)], ir, semL)

    def drain_load(slot):
        buf, ir, semL = slot
        pltpu.make_async_copy(m_hbm.at[pl.ds(0, CH)], buf, semL).wait()
        pltpu.make_async_copy(idx_hbm.at[pl.ds(0, CH)], ir, semL).wait()

    def scatter(slot):
        buf, ir, _ = slot
        pltpu.async_copy(buf, tab.at[ir], semS, add=True).wait()

    load(0, slots[0])

    def body(p, carry):
        i1 = 2 * p + 1
        drain_load(slots[0])

        @pl.when(i1 < n)
        def _():
            load(i1, slots[1])

        scatter(slots[0])

        @pl.when(i1 < n)
        def _():
            drain_load(slots[1])

            @pl.when(i1 + 1 < n)
            def _():
                load(i1 + 1, slots[0])

            scatter(slots[1])

        return carry

    lax.fori_loop(0, (n + 1) // 2, body, 0)


def _scatter2_body(m_hbm, recv_hbm, send_hbm, zeros_hbm, a_hbm, b_hbm,
                   idx0, idx1, buf0, buf1, tab, semL0, semL1, semS):
    c = lax.axis_index("c")
    s = lax.axis_index("s")
    _tab_init_all(zeros_hbm, tab, s)
    n = jnp.where(s < _S_EXTRA, _S_BASE + 1, _S_BASE)
    start = _S_BASE * s + jnp.minimum(s, _S_EXTRA)
    plsc.subcore_barrier()

    @pl.when(c == 0)
    def _():
        _scatter_pipelined(m_hbm, recv_hbm, tab,
                           [(buf0, idx0, semL0), (buf1, idx1, semL1)],
                           semS, start, n)

    @pl.when(c == 1)
    def _():
        _scatter_pipelined(m_hbm, send_hbm, tab,
                           [(buf0, idx0, semL0), (buf1, idx1, semL1)],
                           semS, start, n)

    plsc.subcore_barrier()

    @pl.when(c == 0)
    def _():
        _tab_writeout_full(tab, a_hbm, s)

    @pl.when(c == 1)
    def _():
        _tab_writeout_full(tab, b_hbm, s)


@functools.partial(
    pl.kernel,
    out_type=[jax.ShapeDtypeStruct((N_NODES, LAT), jnp.float32)] * 2,
    mesh=_MESH,
    scratch_types=[
        pltpu.VMEM((CH,), jnp.int32),
        pltpu.VMEM((CH,), jnp.int32),
        pltpu.VMEM((CH, LAT), jnp.float32),
        pltpu.VMEM((CH, LAT), jnp.float32),
        pltpu.VMEM_SHARED((N_NODES, LAT), jnp.float32),
        pltpu.SemaphoreType.DMA,
        pltpu.SemaphoreType.DMA,
        pltpu.SemaphoreType.DMA,
    ],
)
def _sc_scatter2(m_hbm, recv_hbm, send_hbm, zeros_hbm, a_hbm, b_hbm,
                 idx0, idx1, buf0, buf1, tab, semL0, semL1, semS):
    _scatter2_body(m_hbm, recv_hbm, send_hbm, zeros_hbm, a_hbm, b_hbm,
                   idx0, idx1, buf0, buf1, tab, semL0, semL1, semS)


_S1_CHUNKS = N_CHUNKS // NC
_S1_BASE = _S1_CHUNKS // NS
_S1_EXTRA = _S1_CHUNKS - _S1_BASE * NS


def _scatter1_body(m_hbm, recv_hbm, zeros_hbm, a_hbm, b_hbm,
                   idx0, idx1, buf0, buf1, tab, semL0, semL1, semS):
    c = lax.axis_index("c")
    s = lax.axis_index("s")
    _tab_init_all(zeros_hbm, tab, s)
    n = jnp.where(s < _S1_EXTRA, _S1_BASE + 1, _S1_BASE)
    start = c * _S1_CHUNKS + _S1_BASE * s + jnp.minimum(s, _S1_EXTRA)
    plsc.subcore_barrier()
    _scatter_pipelined(m_hbm, recv_hbm, tab,
                       [(buf0, idx0, semL0), (buf1, idx1, semL1)],
                       semS, start, n)
    plsc.subcore_barrier()

    @pl.when(c == 0)
    def _():
        _tab_writeout_full(tab, a_hbm, s)

    @pl.when(c == 1)
    def _():
        _tab_writeout_full(tab, b_hbm, s)


@functools.partial(
    pl.kernel,
    out_type=[jax.ShapeDtypeStruct((N_NODES, LAT), jnp.float32)] * 2,
    mesh=_MESH,
    scratch_types=[
        pltpu.VMEM((CH,), jnp.int32),
        pltpu.VMEM((CH,), jnp.int32),
        pltpu.VMEM((CH, LAT), jnp.float32),
        pltpu.VMEM((CH, LAT), jnp.float32),
        pltpu.VMEM_SHARED((N_NODES, LAT), jnp.float32),
        pltpu.SemaphoreType.DMA,
        pltpu.SemaphoreType.DMA,
        pltpu.SemaphoreType.DMA,
    ],
)
def _sc_scatter1(m_hbm, recv_hbm, zeros_hbm, a_hbm, b_hbm,
                 idx0, idx1, buf0, buf1, tab, semL0, semL1, semS):
    _scatter1_body(m_hbm, recv_hbm, zeros_hbm, a_hbm, b_hbm,
                   idx0, idx1, buf0, buf1, tab, semL0, semL1, semS)


def kernel(V, E, theta, params, senders, receivers, real_node_indices):
    zeros_tab = jnp.zeros((N_NODES, LAT), jnp.float32)
    mask = real_node_indices.astype(jnp.float32).reshape(N_NODES, 1)
    theta2d = theta.reshape(1, -1)
    mp = params['mp']
    wr = [blk['edge'][0][0][LAT:2 * LAT, :] for blk in mp]
    ws = [blk['edge'][0][0][2 * LAT:, :] for blk in mp]

    Vl, Pr, Ps = _tc_enc_nodes(V, params['node_enc'], wr[0], ws[0])
    G1, G2 = _sc_gather(Pr, Ps, receivers, senders)
    M0, El1 = _tc_edge_enc_tail(E, G1, G2, params['edge_enc'], mp[0]['edge'])
    A0, B0 = _sc_scatter2(M0, receivers, senders, zeros_tab)
    IncA, IncB = _sc_scatter1(El1, receivers, zeros_tab)
    Vl, Pr, Ps = _tc_node_tail(Vl, A0, B0, mp[0]['node'], wr[1], ws[1])

    G1, G2 = _sc_gather(Pr, Ps, receivers, senders)
    M1 = _tc_edge_tail_last(El1, G1, G2, mp[1]['edge'])
    A1, B1 = _sc_scatter2(M1, receivers, senders, zeros_tab)

    dec = params['dec']
    dW0t = jnp.concatenate([dec[d][0][0][:LAT, :] for d in range(3)], axis=0)
    dW0z = jnp.concatenate([dec[d][0][0][LAT:, :] for d in range(3)], axis=0)
    db0 = jnp.stack([dec[d][0][1] for d in range(3)])
    dW1 = jnp.concatenate([dec[d][1][0] for d in range(3)], axis=0)
    db1 = jnp.stack([dec[d][1][1] for d in range(3)])
    dW2 = jnp.concatenate([dec[d][2][0] for d in range(3)], axis=1)
    db2 = jnp.stack([dec[d][2][1] for d in range(3)]).reshape(1, 3)

    dbe = _tc_theta(theta2d, params['theta_enc'], dW0t, db0)
    g_f, beta_f = params['final_ln']
    return _tc_node_final(Vl, A1, B1, IncA, IncB, mask, mp[1]['node'],
                          g_f, beta_f, dW0z, dbe, dW1, db1, dW2, db2)


# speedup vs baseline: 4.9675x; 1.0447x over previous
import functools

import jax
import jax.numpy as jnp
from jax import lax
from jax.experimental import pallas as pl
from jax.experimental.pallas import tpu as pltpu
from jax.experimental.pallas import tpu_sc as plsc

N_NODES = 10000
N_EDGES = 160000
LAT = 128
CH = 128
N_CHUNKS = N_EDGES // CH
NC, NS = 2, 16
NW = NC * NS
BLK_N = 2000
BLK_E = 4000
HALF = LAT // 2


def _celu(x):
    return jnp.where(x > 0, x, jnp.exp(jnp.minimum(x, 0.0)) - 1.0)


def _ln(x, g, beta):
    mu = jnp.mean(x, axis=-1, keepdims=True)
    d = x - mu
    var = jnp.mean(d * d, axis=-1, keepdims=True)
    return d * lax.rsqrt(var + 1e-6) * g + beta


def _mlp3(x, W0, b0, W1, b1, W2, b2, g, beta):
    h = _celu(jnp.dot(x, W0, preferred_element_type=jnp.float32) + b0)
    h = _celu(jnp.dot(h, W1, preferred_element_type=jnp.float32) + b1)
    h = jnp.dot(h, W2, preferred_element_type=jnp.float32) + b2
    return _ln(h, g, beta)


def _enc_nodes_body(v, W0, b0, W1, b1, W2, b2, g, beta, wr, ws, vl_o, pr_o, ps_o):
    vl = _mlp3(v[...], W0[...], b0[...], W1[...], b1[...], W2[...], b2[...],
               g[...], beta[...])
    vl_o[...] = vl
    pr_o[...] = jnp.dot(vl, wr[...], preferred_element_type=jnp.float32)
    ps_o[...] = jnp.dot(vl, ws[...], preferred_element_type=jnp.float32)


def _edge_enc_tail_body(e, eW0, eb0, eW1, eb1, eW2, eb2, eg, ebeta,
                        g1, g2, W0a, b0, W1, b1, W2, b2, g, beta, m_o, eln_o):
    el = _mlp3(e[...], eW0[...], eb0[...], eW1[...], eb1[...], eW2[...],
               eb2[...], eg[...], ebeta[...])
    h = _celu(jnp.dot(el, W0a[...], preferred_element_type=jnp.float32)
              + g1[...] + g2[...] + b0[...])
    h = _celu(jnp.dot(h, W1[...], preferred_element_type=jnp.float32) + b1[...])
    h = jnp.dot(h, W2[...], preferred_element_type=jnp.float32) + b2[...]
    m = _ln(h, g[...], beta[...])
    m_o[...] = m
    eln_o[...] = el + m


def _edge_tail_last_body(el, g1, g2, W0a, b0, W1, b1, W2, b2, g, beta, m_o):
    x = el[...]
    h = _celu(jnp.dot(x, W0a[...], preferred_element_type=jnp.float32)
              + g1[...] + g2[...] + b0[...])
    h = _celu(jnp.dot(h, W1[...], preferred_element_type=jnp.float32) + b1[...])
    h = jnp.dot(h, W2[...], preferred_element_type=jnp.float32) + b2[...]
    m_o[...] = _ln(h, g[...], beta[...])


def _node_tail_body(vl, a, b, W0a, W0b, b0, W1, b1, W2, b2, g, beta, wr, ws,
                    vln_o, pr_o, ps_o):
    x = vl[...]
    s = a[...] - b[...]
    h = _celu(jnp.dot(x, W0a[...], preferred_element_type=jnp.float32)
              + jnp.dot(s, W0b[...], preferred_element_type=jnp.float32)
              + b0[...])
    h = _celu(jnp.dot(h, W1[...], preferred_element_type=jnp.float32) + b1[...])
    h = jnp.dot(h, W2[...], preferred_element_type=jnp.float32) + b2[...]
    vln = x + _ln(h, g[...], beta[...])
    vln_o[...] = vln
    if pr_o is not None:
        pr_o[...] = jnp.dot(vln, wr[...], preferred_element_type=jnp.float32)
        ps_o[...] = jnp.dot(vln, ws[...], preferred_element_type=jnp.float32)


def _node_final_body(vl, a, b, inca, incb, mask, W0a, W0b, b0, W1, b1, W2, b2,
                     g, beta, g_f, beta_f, dW0z, dbe, dW1, db1, dW2, db2, out):
    x = vl[...]
    s = a[...] - b[...]
    h = _celu(jnp.dot(x, W0a[...], preferred_element_type=jnp.float32)
              + jnp.dot(s, W0b[...], preferred_element_type=jnp.float32)
              + b0[...])
    h = _celu(jnp.dot(h, W1[...], preferred_element_type=jnp.float32) + b1[...])
    h = jnp.dot(h, W2[...], preferred_element_type=jnp.float32) + b2[...]
    vln = x + _ln(h, g[...], beta[...])
    inc = inca[...] + incb[...] + a[...]
    m = mask[...]
    z = jnp.concatenate([vln * m, inc * m], axis=1)
    zl = _ln(z, g_f[...], beta_f[...])
    cols = []
    for d in range(3):
        hd = _celu(jnp.dot(zl, dW0z[d * 2 * LAT:(d + 1) * 2 * LAT, :],
                           preferred_element_type=jnp.float32) + dbe[d:d + 1, :])
        hd = _celu(jnp.dot(hd, dW1[d * LAT:(d + 1) * LAT, :],
                           preferred_element_type=jnp.float32) + db1[d:d + 1, :])
        cols.append(jnp.dot(hd, dW2[:, d:d + 1],
                            preferred_element_type=jnp.float32))
    out[...] = jnp.concatenate(cols, axis=1) + db2[...]


def _theta_body(t, W0, b0, W1, b1, W2, b2, g, beta, dW0t, db0, out):
    h = _celu(jnp.dot(t[...], W0[...], preferred_element_type=jnp.float32) + b0[...])
    h = _celu(jnp.dot(h, W1[...], preferred_element_type=jnp.float32) + b1[...])
    h = jnp.dot(h, W2[...], preferred_element_type=jnp.float32) + b2[...]
    zt = _ln(h, g[...], beta[...])
    rows = []
    for d in range(3):
        wd = dW0t[d * LAT:(d + 1) * LAT, :]
        rows.append(jnp.dot(zt, wd, preferred_element_type=jnp.float32)
                    + db0[d:d + 1, :])
    out[...] = jnp.concatenate(rows, axis=0)


def _full(shape):
    return pl.BlockSpec(shape, lambda i: (0,) * len(shape))


def _rows(blk, width):
    return pl.BlockSpec((blk, width), lambda i: (i, 0))


def _tc_enc_nodes(V, p, wr, ws):
    (W0, b0), (W1, b1), (W2, b2), (g, beta) = p
    n = N_NODES // BLK_N
    args = [V, W0, b0.reshape(1, -1), W1, b1.reshape(1, -1), W2,
            b2.reshape(1, -1), g.reshape(1, -1), beta.reshape(1, -1), wr, ws]
    specs = [_rows(BLK_N, LAT)] + [_full(a.shape) for a in args[1:]]
    return pl.pallas_call(
        _enc_nodes_body,
        grid=(n,),
        in_specs=specs,
        out_specs=[_rows(BLK_N, LAT)] * 3,
        out_shape=[jax.ShapeDtypeStruct((N_NODES, LAT), jnp.float32)] * 3,
    )(*args)


def _tc_edge_enc_tail(E, G1, G2, enc_p, p):
    (eW0, eb0), (eW1, eb1), (eW2, eb2), (eg, ebeta) = enc_p
    (W0, b0), (W1, b1), (W2, b2), (g, beta) = p
    W0a = W0[:LAT, :]
    n = N_EDGES // BLK_E
    args = [E, eW0, eb0.reshape(1, -1), eW1, eb1.reshape(1, -1), eW2,
            eb2.reshape(1, -1), eg.reshape(1, -1), ebeta.reshape(1, -1),
            G1, G2, W0a, b0.reshape(1, -1), W1, b1.reshape(1, -1), W2,
            b2.reshape(1, -1), g.reshape(1, -1), beta.reshape(1, -1)]
    specs = ([_rows(BLK_E, E.shape[1])]
             + [_full(a.shape) for a in args[1:9]]
             + [_rows(BLK_E, LAT)] * 2
             + [_full(a.shape) for a in args[11:]])
    return pl.pallas_call(
        _edge_enc_tail_body,
        grid=(n,),
        in_specs=specs,
        out_specs=[_rows(BLK_E, LAT)] * 2,
        out_shape=[jax.ShapeDtypeStruct((N_EDGES, LAT), jnp.float32)] * 2,
    )(*args)


def _tc_edge_tail_last(El, G1, G2, p):
    (W0, b0), (W1, b1), (W2, b2), (g, beta) = p
    W0a = W0[:LAT, :]
    n = N_EDGES // BLK_E
    args = [El, G1, G2, W0a, b0.reshape(1, -1), W1, b1.reshape(1, -1), W2,
            b2.reshape(1, -1), g.reshape(1, -1), beta.reshape(1, -1)]
    specs = [_rows(BLK_E, LAT)] * 3 + [_full(a.shape) for a in args[3:]]
    return pl.pallas_call(
        _edge_tail_last_body,
        grid=(n,),
        in_specs=specs,
        out_specs=_rows(BLK_E, LAT),
        out_shape=jax.ShapeDtypeStruct((N_EDGES, LAT), jnp.float32),
    )(*args)


def _tc_node_tail(Vl, A, B, p, wr, ws):
    (W0, b0), (W1, b1), (W2, b2), (g, beta) = p
    W0a, W0b = W0[:LAT, :], W0[LAT:, :]
    n = N_NODES // BLK_N
    args = [Vl, A, B, W0a, W0b, b0.reshape(1, -1), W1, b1.reshape(1, -1), W2,
            b2.reshape(1, -1), g.reshape(1, -1), beta.reshape(1, -1), wr, ws]
    specs = ([_rows(BLK_N, LAT)] * 3
             + [_full(a.shape) for a in args[3:]])
    return pl.pallas_call(
        _node_tail_body,
        grid=(n,),
        in_specs=specs,
        out_specs=[_rows(BLK_N, LAT)] * 3,
        out_shape=[jax.ShapeDtypeStruct((N_NODES, LAT), jnp.float32)] * 3,
    )(*args)


def _tc_node_final(Vl, A, B, IncA, IncB, mask, p, g_f, beta_f,
                   dW0z, dbe, dW1, db1, dW2, db2):
    (W0, b0), (W1, b1), (W2, b2), (g, beta) = p
    W0a, W0b = W0[:LAT, :], W0[LAT:, :]
    n = N_NODES // BLK_N
    args = [Vl, A, B, IncA, IncB, mask, W0a, W0b, b0.reshape(1, -1),
            W1, b1.reshape(1, -1), W2, b2.reshape(1, -1),
            g.reshape(1, -1), beta.reshape(1, -1),
            g_f.reshape(1, -1), beta_f.reshape(1, -1),
            dW0z, dbe, dW1, db1, dW2, db2]
    specs = ([_rows(BLK_N, LAT)] * 5 + [_rows(BLK_N, 1)]
             + [_full(a.shape) for a in args[6:]])
    return pl.pallas_call(
        _node_final_body,
        grid=(n,),
        in_specs=specs,
        out_specs=_rows(BLK_N, 3),
        out_shape=jax.ShapeDtypeStruct((N_NODES, 3), jnp.float32),
    )(*args)


def _tc_theta(theta2d, p, dW0t, db0):
    (W0, b0), (W1, b1), (W2, b2), (g, beta) = p
    args = [theta2d, W0, b0.reshape(1, -1), W1, b1.reshape(1, -1), W2,
            b2.reshape(1, -1), g.reshape(1, -1), beta.reshape(1, -1), dW0t, db0]
    return pl.pallas_call(
        _theta_body,
        grid=(1,),
        in_specs=[_full(a.shape) for a in args],
        out_specs=_full((3, LAT)),
        out_shape=jax.ShapeDtypeStruct((3, LAT), jnp.float32),
    )(*args)


_MESH = plsc.VectorSubcoreMesh(core_axis_name="c", subcore_axis_name="s")

_G_BASE = N_CHUNKS // NW
_G_EXTRA = N_CHUNKS - _G_BASE * NW
_S_BASE = N_CHUNKS // NS
_S_EXTRA = N_CHUNKS - _S_BASE * NS
_TAB_SPAN = 624


def _gather_body(pr_hbm, ps_hbm, recv_hbm, send_hbm, g1_hbm, g2_hbm,
                 idxr0, idxs0, idxr1, idxs1, bufr0, bufs0, bufr1, bufs1,
                 semg0, semg1, semw0, semw1):
    c = lax.axis_index("c")
    s = lax.axis_index("s")
    w = s * NC + c
    nw = jnp.where(w < _G_EXTRA, _G_BASE + 1, _G_BASE)
    start = _G_BASE * w + jnp.minimum(w, _G_EXTRA)

    def load_idx(i, ir, is_):
        off = (start + i) * CH
        pltpu.sync_copy(recv_hbm.at[pl.ds(off, CH)], ir)
        pltpu.sync_copy(send_hbm.at[pl.ds(off, CH)], is_)

    def body(p, carry):
        i0 = 2 * p
        i1 = i0 + 1
        load_idx(i0, idxr0, idxs0)
        dg0a = pltpu.async_copy(pr_hbm.at[idxr0], bufr0, semg0)
        dg0b = pltpu.async_copy(ps_hbm.at[idxs0], bufs0, semg0)

        @pl.when(i1 < nw)
        def _():
            load_idx(i1, idxr1, idxs1)
            dg1a = pltpu.async_copy(pr_hbm.at[idxr1], bufr1, semg1)
            dg1b = pltpu.async_copy(ps_hbm.at[idxs1], bufs1, semg1)
            dg0a.wait()
            dg0b.wait()
            off0 = (i0 + start) * CH
            dw0a = pltpu.async_copy(bufr0, g1_hbm.at[pl.ds(off0, CH)], semw0)
            dw0b = pltpu.async_copy(bufs0, g2_hbm.at[pl.ds(off0, CH)], semw0)
            dg1a.wait()
            dg1b.wait()
            off1 = (i1 + start) * CH
            dw1a = pltpu.async_copy(bufr1, g1_hbm.at[pl.ds(off1, CH)], semw1)
            dw1b = pltpu.async_copy(bufs1, g2_hbm.at[pl.ds(off1, CH)], semw1)
            dw0a.wait()
            dw0b.wait()
            dw1a.wait()
            dw1b.wait()

        @pl.when(i1 >= nw)
        def _():
            dg0a.wait()
            dg0b.wait()
            off0 = (i0 + start) * CH
            dw0a = pltpu.async_copy(bufr0, g1_hbm.at[pl.ds(off0, CH)], semw0)
            dw0b = pltpu.async_copy(bufs0, g2_hbm.at[pl.ds(off0, CH)], semw0)
            dw0a.wait()
            dw0b.wait()

        return carry

    lax.fori_loop(0, (nw + 1) // 2, body, 0)


@functools.partial(
    pl.kernel,
    out_type=[jax.ShapeDtypeStruct((N_EDGES, LAT), jnp.float32)] * 2,
    mesh=_MESH,
    scratch_types=[
        pltpu.VMEM((CH,), jnp.int32),
        pltpu.VMEM((CH,), jnp.int32),
        pltpu.VMEM((CH,), jnp.int32),
        pltpu.VMEM((CH,), jnp.int32),
        pltpu.VMEM((CH, LAT), jnp.float32),
        pltpu.VMEM((CH, LAT), jnp.float32),
        pltpu.VMEM((CH, LAT), jnp.float32),
        pltpu.VMEM((CH, LAT), jnp.float32),
        pltpu.SemaphoreType.DMA,
        pltpu.SemaphoreType.DMA,
        pltpu.SemaphoreType.DMA,
        pltpu.SemaphoreType.DMA,
    ],
)
def _sc_gather(pr_hbm, ps_hbm, recv_hbm, send_hbm, g1_hbm, g2_hbm,
               idxr0, idxs0, idxr1, idxs1, bufr0, bufs0, bufr1, bufs1,
               semg0, semg1, semw0, semw1):
    _gather_body(pr_hbm, ps_hbm, recv_hbm, send_hbm, g1_hbm, g2_hbm,
                 idxr0, idxs0, idxr1, idxs1, bufr0, bufs0, bufr1, bufs1,
                 semg0, semg1, semw0, semw1)


def _tab_init_all(zeros_hbm, table, s):
    r0 = s * _TAB_SPAN
    pltpu.sync_copy(zeros_hbm.at[pl.ds(r0, _TAB_SPAN)],
                    table.at[pl.ds(r0, _TAB_SPAN)])

    @pl.when(s == NS - 1)
    def _():
        tail = N_NODES - NS * _TAB_SPAN
        pltpu.sync_copy(zeros_hbm.at[pl.ds(NS * _TAB_SPAN, tail)],
                        table.at[pl.ds(NS * _TAB_SPAN, tail)])


def _tab_writeout_full(table, out_hbm, s):
    r0 = s * _TAB_SPAN
    pltpu.sync_copy(table.at[pl.ds(r0, _TAB_SPAN)],
                    out_hbm.at[pl.ds(r0, _TAB_SPAN)])

    @pl.when(s == NS - 1)
    def _():
        tail = N_NODES - NS * _TAB_SPAN
        pltpu.sync_copy(table.at[pl.ds(NS * _TAB_SPAN, tail)],
                        out_hbm.at[pl.ds(NS * _TAB_SPAN, tail)])


def _scatter_pipelined(m_hbm, idx_hbm, tab, slots, semS, start, n):
    def load(i, slot):
        buf, ir, semL = slot
        off = (start + i) * CH
        pltpu.async_copy(m_hbm.at[pl.ds(off, CH)], buf, semL)
        pltpu.async_copy(idx_hbm.at[pl.ds(off, CH)], ir, semL)

    def drain_load(slot):
        buf, ir, semL = slot
        pltpu.make_async_copy(m_hbm.at[pl.ds(0, CH)], buf, semL).wait()
        pltpu.make_async_copy(idx_hbm.at[pl.ds(0, CH)], ir, semL).wait()

    def scatter(slot):
        buf, ir, _ = slot
        pltpu.async_copy(buf, tab.at[ir], semS, add=True).wait()

    load(0, slots[0])

    def body(p, carry):
        i1 = 2 * p + 1
        drain_load(slots[0])

        @pl.when(i1 < n)
        def _():
            load(i1, slots[1])

        scatter(slots[0])

        @pl.when(i1 < n)
        def _():
            drain_load(slots[1])

            @pl.when(i1 + 1 < n)
            def _():
                load(i1 + 1, slots[0])

            scatter(slots[1])

        return carry

    lax.fori_loop(0, (n + 1) // 2, body, 0)


def _scatter2_body(m_hbm, recv_hbm, send_hbm, zeros_hbm, a_hbm, b_hbm,
                   idx0, idx1, buf0, buf1, tab, semL0, semL1, semS):
    c = lax.axis_index("c")
    s = lax.axis_index("s")
    _tab_init_all(zeros_hbm, tab, s)
    n = jnp.where(s < _S_EXTRA, _S_BASE + 1, _S_BASE)
    start = _S_BASE * s + jnp.minimum(s, _S_EXTRA)
    plsc.subcore_barrier()

    @pl.when(c == 0)
    def _():
        _scatter_pipelined(m_hbm, recv_hbm, tab,
                           [(buf0, idx0, semL0), (buf1, idx1, semL1)],
                           semS, start, n)

    @pl.when(c == 1)
    def _():
        _scatter_pipelined(m_hbm, send_hbm, tab,
                           [(buf0, idx0, semL0), (buf1, idx1, semL1)],
                           semS, start, n)

    plsc.subcore_barrier()

    @pl.when(c == 0)
    def _():
        _tab_writeout_full(tab, a_hbm, s)

    @pl.when(c == 1)
    def _():
        _tab_writeout_full(tab, b_hbm, s)


@functools.partial(
    pl.kernel,
    out_type=[jax.ShapeDtypeStruct((N_NODES, LAT), jnp.float32)] * 2,
    mesh=_MESH,
    scratch_types=[
        pltpu.VMEM((CH,), jnp.int32),
        pltpu.VMEM((CH,), jnp.int32),
        pltpu.VMEM((CH, LAT), jnp.float32),
        pltpu.VMEM((CH, LAT), jnp.float32),
        pltpu.VMEM_SHARED((N_NODES, LAT), jnp.float32),
        pltpu.SemaphoreType.DMA,
        pltpu.SemaphoreType.DMA,
        pltpu.SemaphoreType.DMA,
    ],
)
def _sc_scatter2(m_hbm, recv_hbm, send_hbm, zeros_hbm, a_hbm, b_hbm,
                 idx0, idx1, buf0, buf1, tab, semL0, semL1, semS):
    _scatter2_body(m_hbm, recv_hbm, send_hbm, zeros_hbm, a_hbm, b_hbm,
                   idx0, idx1, buf0, buf1, tab, semL0, semL1, semS)


_S1_CHUNKS = N_CHUNKS // NC
_S1_BASE = _S1_CHUNKS // NS
_S1_EXTRA = _S1_CHUNKS - _S1_BASE * NS


def _scatter1_body(m_hbm, recv_hbm, zeros_hbm, a_hbm, b_hbm,
                   idx0, idx1, buf0, buf1, tab, semL0, semL1, semS):
    c = lax.axis_index("c")
    s = lax.axis_index("s")
    _tab_init_all(zeros_hbm, tab, s)
    n = jnp.where(s < _S1_EXTRA, _S1_BASE + 1, _S1_BASE)
    start = c * _S1_CHUNKS + _S1_BASE * s + jnp.minimum(s, _S1_EXTRA)
    plsc.subcore_barrier()
    _scatter_pipelined(m_hbm, recv_hbm, tab,
                       [(buf0, idx0, semL0), (buf1, idx1, semL1)],
                       semS, start, n)
    plsc.subcore_barrier()

    @pl.when(c == 0)
    def _():
        _tab_writeout_full(tab, a_hbm, s)

    @pl.when(c == 1)
    def _():
        _tab_writeout_full(tab, b_hbm, s)


@functools.partial(
    pl.kernel,
    out_type=[jax.ShapeDtypeStruct((N_NODES, LAT), jnp.float32)] * 2,
    mesh=_MESH,
    scratch_types=[
        pltpu.VMEM((CH,), jnp.int32),
        pltpu.VMEM((CH,), jnp.int32),
        pltpu.VMEM((CH, LAT), jnp.float32),
        pltpu.VMEM((CH, LAT), jnp.float32),
        pltpu.VMEM_SHARED((N_NODES, LAT), jnp.float32),
        pltpu.SemaphoreType.DMA,
        pltpu.SemaphoreType.DMA,
        pltpu.SemaphoreType.DMA,
    ],
)
def _sc_scatter1(m_hbm, recv_hbm, zeros_hbm, a_hbm, b_hbm,
                 idx0, idx1, buf0, buf1, tab, semL0, semL1, semS):
    _scatter1_body(m_hbm, recv_hbm, zeros_hbm, a_hbm, b_hbm,
                   idx0, idx1, buf0, buf1, tab, semL0, semL1, semS)


def kernel(V, E, theta, params, senders, receivers, real_node_indices):
    zeros_tab = jnp.zeros((N_NODES, LAT), jnp.float32)
    mask = real_node_indices.astype(jnp.float32).reshape(N_NODES, 1)
    theta2d = theta.reshape(1, -1)
    mp = params['mp']
    wr = [blk['edge'][0][0][LAT:2 * LAT, :] for blk in mp]
    ws = [blk['edge'][0][0][2 * LAT:, :] for blk in mp]

    Vl, Pr, Ps = _tc_enc_nodes(V, params['node_enc'], wr[0], ws[0])
    G1, G2 = _sc_gather(Pr, Ps, receivers, senders)
    M0, El1 = _tc_edge_enc_tail(E, G1, G2, params['edge_enc'], mp[0]['edge'])
    A0, B0 = _sc_scatter2(M0, receivers, senders, zeros_tab)
    IncA, IncB = _sc_scatter1(El1, receivers, zeros_tab)
    Vl, Pr, Ps = _tc_node_tail(Vl, A0, B0, mp[0]['node'], wr[1], ws[1])

    G1, G2 = _sc_gather(Pr, Ps, receivers, senders)
    M1 = _tc_edge_tail_last(El1, G1, G2, mp[1]['edge'])
    A1, B1 = _sc_scatter2(M1, receivers, senders, zeros_tab)

    dec = params['dec']
    dW0t = jnp.concatenate([dec[d][0][0][:LAT, :] for d in range(3)], axis=0)
    dW0z = jnp.concatenate([dec[d][0][0][LAT:, :] for d in range(3)], axis=0)
    db0 = jnp.stack([dec[d][0][1] for d in range(3)])
    dW1 = jnp.concatenate([dec[d][1][0] for d in range(3)], axis=0)
    db1 = jnp.stack([dec[d][1][1] for d in range(3)])
    dW2 = jnp.concatenate([dec[d][2][0] for d in range(3)], axis=1)
    db2 = jnp.stack([dec[d][2][1] for d in range(3)]).reshape(1, 3)

    dbe = _tc_theta(theta2d, params['theta_enc'], dW0t, db0)
    g_f, beta_f = params['final_ln']
    return _tc_node_final(Vl, A1, B1, IncA, IncB, mask, mp[1]['node'],
                          g_f, beta_f, dW0z, dbe, dW1, db1, dW2, db2)
